# baseline (speedup 1.0000x reference)
import functools

import jax
import jax.numpy as jnp
from jax.experimental import pallas as pl

N_NODES = 10000
HEADS = 4
HID = 64


def _mm_body(x_ref, w_ref, o_ref):
    o_ref[...] = jnp.dot(x_ref[...], w_ref[...],
                         preferred_element_type=jnp.float32)


@functools.partial(jax.jit, static_argnames=())
def _pallas_matmul(x, w):
    n, k = x.shape
    m = w.shape[1]
    bn = 1000
    grid = (n // bn,)
    return pl.pallas_call(
        _mm_body,
        grid=grid,
        in_specs=[
            pl.BlockSpec((bn, k), lambda i: (i, 0)),
            pl.BlockSpec((k, m), lambda i: (0, 0)),
        ],
        out_specs=pl.BlockSpec((bn, m), lambda i: (i, 0)),
        out_shape=jax.ShapeDtypeStruct((n, m), jnp.float32),
    )(x, w)


def _gat_layer(x, src, dst, W, a_src, a_dst, b, heads, out_ch, concat):
    N = x.shape[0]
    h = _pallas_matmul(x, W).reshape(N, heads, out_ch)
    alpha_src = jnp.sum(h * a_src, axis=-1)
    alpha_dst = jnp.sum(h * a_dst, axis=-1)
    alpha = alpha_src[src] + alpha_dst[dst]
    alpha = jax.nn.leaky_relu(alpha, negative_slope=0.2)
    amax = jax.ops.segment_max(alpha, dst, num_segments=N)
    ex = jnp.exp(alpha - amax[dst])
    denom = jax.ops.segment_sum(ex, dst, num_segments=N)
    attn = ex / (denom[dst] + 1e-16)
    msg = h[src] * attn[:, :, None]
    out = jax.ops.segment_sum(msg, dst, num_segments=N)
    if concat:
        out = out.reshape(N, heads * out_ch)
    else:
        out = out.mean(axis=1)
    return out + b


def kernel(x, edge_index, W1, a_src1, a_dst1, b1, W2, a_src2, a_dst2, b2, W3, a_src3, a_dst3, b3):
    N = x.shape[0]
    loop = jnp.arange(N, dtype=edge_index.dtype)
    src = jnp.concatenate([edge_index[0], loop])
    dst = jnp.concatenate([edge_index[1], loop])
    h = jax.nn.elu(_gat_layer(x, src, dst, W1, a_src1, a_dst1, b1, HEADS, HID, True))
    h = jax.nn.elu(_gat_layer(h, src, dst, W2, a_src2, a_dst2, b2, HEADS, HID, True))
    node_embeddings = _gat_layer(h, src, dst, W3, a_src3, a_dst3, b3, 1, HID, False)
    graph_embedding = jnp.mean(node_embeddings, axis=0, keepdims=True)
    return (node_embeddings, graph_embedding)


# speedup vs baseline: 27.4808x; 27.4808x over previous
import functools

import jax
import jax.numpy as jnp
from jax import lax
from jax.experimental import pallas as pl
from jax.experimental.pallas import tpu as pltpu
from jax.experimental.pallas import tpu_sc as plsc

N = 10000
NP = 10240
E_REAL = 650000
EPAD = 655360
CHUNK = 128
HEADS = 4
HID = 64
F32 = jnp.float32
I32 = jnp.int32


def _elu(x):
    return jnp.where(x > 0, x, jnp.exp(x) - 1.0)


def _leaky(x):
    return jnp.where(x > 0, x, 0.2 * x)


def _a1_body(x_ref, w_ref, aat_ref, h0_ref, h1_ref, h2_ref, h3_ref, aux_ref):
    h = jnp.dot(x_ref[...], w_ref[...], preferred_element_type=F32)
    h0_ref[...] = h[:, 0:64]
    h1_ref[...] = h[:, 64:128]
    h2_ref[...] = h[:, 128:192]
    h3_ref[...] = h[:, 192:256]
    aux_ref[...] = jnp.dot(h, aat_ref[...], preferred_element_type=F32)


def _tc_layer1(xpad, W1, Aat1):
    bn = 1024
    hspec = pl.BlockSpec((bn, 64), lambda i: (i, 0))
    hshape = jax.ShapeDtypeStruct((NP, 64), F32)
    return pl.pallas_call(
        _a1_body,
        grid=(NP // bn,),
        in_specs=[
            pl.BlockSpec((bn, 16), lambda i: (i, 0)),
            pl.BlockSpec((16, 256), lambda i: (0, 0)),
            pl.BlockSpec((256, 8), lambda i: (0, 0)),
        ],
        out_specs=[hspec, hspec, hspec, hspec,
                   pl.BlockSpec((bn, 8), lambda i: (i, 0))],
        out_shape=[hshape, hshape, hshape, hshape,
                   jax.ShapeDtypeStruct((NP, 8), F32)],
    )(xpad, W1, Aat1)


def _a_next_body(n0_ref, n1_ref, n2_ref, n3_ref, den_ref, b_ref, w_ref,
                 aat_ref, *out_refs, bn, out_ch):
    b = b_ref[...]
    den = den_ref[...]
    nq = (n0_ref, n1_ref, n2_ref, n3_ref)
    xs = []
    for q in range(4):
        dq = jnp.broadcast_to(den[:, q:q + 1] + 1e-16, (bn, 64))
        xs.append(_elu(nq[q][...] / dq + b[:, 64 * q:64 * (q + 1)]))
    x = jnp.concatenate(xs, axis=1)
    h = jnp.dot(x, w_ref[...], preferred_element_type=F32)
    if out_ch == 256:
        h0_ref, h1_ref, h2_ref, h3_ref, aux_ref = out_refs
        h0_ref[...] = h[:, 0:64]
        h1_ref[...] = h[:, 64:128]
        h2_ref[...] = h[:, 128:192]
        h3_ref[...] = h[:, 192:256]
    else:
        h0_ref, aux_ref = out_refs
        h0_ref[...] = h
    aux_ref[...] = jnp.dot(h, aat_ref[...], preferred_element_type=F32)


def _tc_layer_next(n0, n1, n2, n3, den, brow, W, Aat, out_ch):
    bn = 1024
    body = functools.partial(_a_next_body, bn=bn, out_ch=out_ch)
    hspec = pl.BlockSpec((bn, 64), lambda i: (i, 0))
    hshape = jax.ShapeDtypeStruct((NP, 64), F32)
    if out_ch == 256:
        out_specs = [hspec, hspec, hspec, hspec,
                     pl.BlockSpec((bn, 8), lambda i: (i, 0))]
        out_shape = [hshape, hshape, hshape, hshape,
                     jax.ShapeDtypeStruct((NP, 8), F32)]
    else:
        out_specs = [hspec, pl.BlockSpec((bn, 8), lambda i: (i, 0))]
        out_shape = [hshape, jax.ShapeDtypeStruct((NP, 8), F32)]
    return pl.pallas_call(
        body,
        grid=(NP // bn,),
        in_specs=[
            hspec, hspec, hspec, hspec,
            pl.BlockSpec((bn, 128), lambda i: (i, 0)),
            pl.BlockSpec((1, 256), lambda i: (0, 0)),
            pl.BlockSpec((256, W.shape[1]), lambda i: (0, 0)),
            pl.BlockSpec((W.shape[1], 8), lambda i: (0, 0)),
        ],
        out_specs=out_specs,
        out_shape=out_shape,
    )(n0, n1, n2, n3, den, brow, W, Aat)


def _c_body(na_ref, nb_ref, da_ref, db_ref, b_ref, ne_ref, g_ref):
    i = pl.program_id(0)
    bn = 1000
    den = (da_ref[...][:, 0:1] + db_ref[...][:, 0:1]) + 1e-16
    ne = (na_ref[...] + nb_ref[...]) / jnp.broadcast_to(den, (bn, 64)) \
        + b_ref[...]
    ne_ref[...] = ne

    @pl.when(i == 0)
    def _():
        g_ref[...] = jnp.zeros((1, 64), F32)

    g_ref[...] += jnp.sum(ne, axis=0, keepdims=True) * (1.0 / N)


def _tc_final(num3A, num3B, den3A, den3B, b3row):
    bn = 1000
    return pl.pallas_call(
        _c_body,
        grid=(N // bn,),
        in_specs=[
            pl.BlockSpec((bn, 64), lambda i: (i, 0)),
            pl.BlockSpec((bn, 64), lambda i: (i, 0)),
            pl.BlockSpec((bn, 128), lambda i: (i, 0)),
            pl.BlockSpec((bn, 128), lambda i: (i, 0)),
            pl.BlockSpec((1, 64), lambda i: (0, 0)),
        ],
        out_specs=[
            pl.BlockSpec((bn, 64), lambda i: (i, 0)),
            pl.BlockSpec((1, 64), lambda i: (0, 0)),
        ],
        out_shape=[
            jax.ShapeDtypeStruct((N, 64), F32),
            jax.ShapeDtypeStruct((1, 64), F32),
        ],
    )(num3A, num3B, den3A, den3B, b3row)


def _splat(v):
    return lax.broadcast_in_dim(jnp.asarray(v, I32), (16,), ())


def _sc_edge_body(src_ref, dst_ref, auxs_ref, auxdm_ref, h_ref,
                  num_ref, den_ref,
                  idx_s, idx_d, hidx, auxs, auxdm, rows, exflat0, exflat1,
                  exrow, dbuf,
                  zbuf, zbuf_d, acc_sh, den_sh, gsem, ssem,
                  *, qj):
    cid = lax.axis_index("c")
    sid = lax.axis_index("s")
    chan = 64
    nlanes = chan // 16

    if qj is None:
        n_tiles = 32
        tile_id = sid * 2 + cid
        h_off = jnp.asarray(0, I32)
        head_base = jnp.asarray(0, I32)
    else:
        n_tiles = 16
        tile_id = sid
        h_off = (2 * qj + cid) * NP
        head_base = 2 * qj + cid

    epw = EPAD // n_tiles
    nch = epw // CHUNK
    e0 = tile_id * epw

    def zrow(r, _):
        for p in range(nlanes):
            zbuf[r, pl.ds(16 * p, 16)] = jnp.zeros((16,), F32)
        zbuf_d[r, pl.ds(0, 16)] = jnp.zeros((16,), F32)
        return 0
    lax.fori_loop(0, 64, zrow, 0)
    rows_per_tile = NP // 16
    r0 = sid * rows_per_tile

    def zcp(j, _):
        pltpu.sync_copy(zbuf, acc_sh.at[pl.ds(r0 + j * 64, 64)])
        pltpu.sync_copy(zbuf_d, den_sh.at[pl.ds(r0 + j * 64, 64)])
        return 0
    lax.fori_loop(0, rows_per_tile // 64, zcp, 0)

    plsc.subcore_barrier()

    def load_idx(c, b):
        pltpu.sync_copy(src_ref.at[pl.ds(e0 + c * CHUNK, CHUNK)],
                        idx_s.at[b])
        pltpu.sync_copy(dst_ref.at[pl.ds(e0 + c * CHUNK, CHUNK)],
                        idx_d.at[b])
        for k in range(8):
            hidx[b, pl.ds(16 * k, 16)] = idx_s[b, pl.ds(16 * k, 16)] + h_off

    def issue_gathers(b):
        pltpu.async_copy(auxs_ref.at[idx_s.at[b]], auxs.at[b], gsem)
        pltpu.async_copy(auxdm_ref.at[idx_d.at[b]], auxdm.at[b], gsem)
        pltpu.async_copy(h_ref.at[hidx.at[b]], rows.at[b], gsem)

    def wait_gathers(b):
        pltpu.make_async_copy(auxs_ref.at[idx_s.at[b]], auxs.at[b],
                              gsem).wait()
        pltpu.make_async_copy(auxdm_ref.at[idx_d.at[b]], auxdm.at[b],
                              gsem).wait()
        pltpu.make_async_copy(h_ref.at[hidx.at[b]], rows.at[b], gsem).wait()

    def issue_scatters(b):
        pltpu.async_copy(rows.at[b], acc_sh.at[idx_d.at[b]], ssem, add=True)
        pltpu.async_copy(exrow.at[b], den_sh.at[idx_d.at[b]], ssem, add=True)

    def wait_scatters(b):
        pltpu.make_async_copy(rows.at[b], acc_sh.at[idx_d.at[b]], ssem).wait()
        pltpu.make_async_copy(exrow.at[b], den_sh.at[idx_d.at[b]], ssem).wait()

    def compute(b):
        exflat = (exflat0, exflat1)[b]

        def mul_body(e, _):
            va = auxs[b, e, pl.ds(0, 16)]
            vd = auxdm[b, e, pl.ds(0, 16)]
            vm = auxdm[b, e, pl.ds(16, 16)]
            t = va + vd
            t = jnp.maximum(t, 0.2 * t)
            ex16 = jnp.exp(t - vm)
            exrow[b, e, pl.ds(0, 16)] = ex16
            exflat[pl.ds(16 * e, 16)] = ex16
            sf = plsc.load_gather(exflat, [_splat(16 * e + head_base)])
            for p in range(4):
                c0 = 16 * p
                rows[b, e, pl.ds(c0, 16)] = rows[b, e, pl.ds(c0, 16)] * sf
            return 0
        lax.fori_loop(0, CHUNK, mul_body, 0)

    load_idx(jnp.asarray(0, I32), 0)
    issue_gathers(0)

    def pair_body(g2, _):
        for b in (0, 1):
            c = 2 * g2 + b
            nb = 1 - b

            @pl.when(c >= 1)
            def _():
                wait_scatters(nb)

            @pl.when(c + 1 < nch)
            def _():
                load_idx(c + 1, nb)
                issue_gathers(nb)

            wait_gathers(b)
            compute(b)
            issue_scatters(b)
        return 0
    lax.fori_loop(0, nch // 2, pair_body, 0)
    wait_scatters(1)
    plsc.subcore_barrier()

    out_base = cid * NP + r0
    rbuf = rows.at[0]

    def wb(j, _):
        pltpu.sync_copy(acc_sh.at[pl.ds(r0 + j * 64, 64)],
                        rbuf.at[pl.ds(0, 64)])
        pltpu.sync_copy(rbuf.at[pl.ds(0, 64)],
                        num_ref.at[pl.ds(out_base + j * 64, 64)])
        return 0
    lax.fori_loop(0, rows_per_tile // 64, wb, 0)

    def wbd(j, _):
        pltpu.sync_copy(den_sh.at[pl.ds(r0 + j * 128, 128)],
                        dbuf.at[pl.ds(0, 128), pl.ds(0, 16)])
        pltpu.sync_copy(dbuf, den_ref.at[pl.ds(out_base + j * 128, 128)])
        return 0
    lax.fori_loop(0, rows_per_tile // 128, wbd, 0)


def _make_sc_edge(qj):
    mesh = plsc.VectorSubcoreMesh(core_axis_name="c", subcore_axis_name="s",
                                  num_cores=2, num_subcores=16)
    body = functools.partial(_sc_edge_body, qj=qj)
    chan = 64
    return pl.kernel(
        body,
        compiler_params=pltpu.CompilerParams(needs_layout_passes=False,
                                             use_tc_tiling_on_sc=False),
        out_type=(
            jax.ShapeDtypeStruct((2 * NP, chan), F32),
            jax.ShapeDtypeStruct((2 * NP, 128), F32),
        ),
        mesh=mesh,
        scratch_types=[
            pltpu.VMEM((2, CHUNK), I32),
            pltpu.VMEM((2, CHUNK), I32),
            pltpu.VMEM((2, CHUNK), I32),
            pltpu.VMEM((2, CHUNK, 16), F32),
            pltpu.VMEM((2, CHUNK, 32), F32),
            pltpu.VMEM((2, CHUNK, chan), F32),
            pltpu.VMEM((CHUNK * 16,), F32),
            pltpu.VMEM((CHUNK * 16,), F32),
            pltpu.VMEM((2, CHUNK, 16), F32),
            pltpu.VMEM((CHUNK, 128), F32),
            pltpu.VMEM((64, chan), F32),
            pltpu.VMEM((64, 16), F32),
            pltpu.VMEM_SHARED((NP, chan), F32),
            pltpu.VMEM_SHARED((NP, 16), F32),
            pltpu.SemaphoreType.DMA,
            pltpu.SemaphoreType.DMA,
        ],
    )


def _head_mats(a_src, a_dst, heads, C):
    k = heads * C
    eye = jnp.eye(heads, dtype=F32)
    Asrc = (eye[:, None, :] * a_src[0][:, :, None]).reshape(k, heads)
    Adst = (eye[:, None, :] * a_dst[0][:, :, None]).reshape(k, heads)
    if heads < 4:
        Asrc = jnp.pad(Asrc, ((0, 0), (0, 4 - heads)))
        Adst = jnp.pad(Adst, ((0, 0), (0, 4 - heads)))
    return jnp.concatenate([Asrc, Adst], axis=1)


def _aux_tables(aux8):
    as_ = aux8[:, :4]
    ad_ = aux8[:, 4:8]
    asmax = jnp.max(as_, axis=0)
    M = _leaky(asmax[None, :] + ad_)
    auxS = jnp.tile(as_, (1, 4))
    auxDM = jnp.concatenate([jnp.tile(ad_, (1, 4)),
                             jnp.tile(M, (1, 4))], axis=1)
    return auxS, auxDM


def kernel(x, edge_index, W1, a_src1, a_dst1, b1, W2, a_src2, a_dst2, b2,
           W3, a_src3, a_dst3, b3):
    loop = jnp.arange(N, dtype=jnp.int32)
    src = jnp.concatenate([edge_index[0].astype(I32), loop])
    dst = jnp.concatenate([edge_index[1].astype(I32), loop])
    padv = jnp.full((EPAD - E_REAL,), N, I32)
    src = jnp.concatenate([src, padv])
    dst = jnp.concatenate([dst, padv])
    xpad = jnp.pad(x, ((0, NP - N), (0, 0)))

    Aat1 = _head_mats(a_src1, a_dst1, HEADS, HID)
    Aat2 = _head_mats(a_src2, a_dst2, HEADS, HID)
    Aat3 = _head_mats(a_src3, a_dst3, 1, HID)
    b1r = b1.reshape(1, 256)
    b2r = b2.reshape(1, 256)
    b3r = b3.reshape(1, 64)

    scq0 = _make_sc_edge(0)
    scq1 = _make_sc_edge(1)
    sc3 = _make_sc_edge(None)

    h0, h1, h2, h3_, aux8 = _tc_layer1(xpad, W1, Aat1)
    auxS, auxDM = _aux_tables(aux8)
    h_all = jnp.concatenate([h0, h1, h2, h3_], axis=0)
    n01, d01 = scq0(src, dst, auxS, auxDM, h_all)
    n23, _ = scq1(src, dst, auxS, auxDM, h_all)

    h0, h1, h2, h3_, aux8 = _tc_layer_next(
        n01[:NP], n01[NP:], n23[:NP], n23[NP:], d01[:NP], b1r, W2, Aat2, 256)
    auxS, auxDM = _aux_tables(aux8)
    h_all = jnp.concatenate([h0, h1, h2, h3_], axis=0)
    n01, d01 = scq0(src, dst, auxS, auxDM, h_all)
    n23, _ = scq1(src, dst, auxS, auxDM, h_all)

    hL3, aux8 = _tc_layer_next(
        n01[:NP], n01[NP:], n23[:NP], n23[NP:], d01[:NP], b2r, W3, Aat3, 64)
    auxS, auxDM = _aux_tables(aux8)
    num3, den3 = sc3(src, dst, auxS, auxDM, hL3)

    node_embeddings, graph_embedding = _tc_final(
        num3[:NP], num3[NP:], den3[:NP], den3[NP:], b3r)
    return (node_embeddings, graph_embedding)


# speedup vs baseline: 35.8753x; 1.3055x over previous
import functools

import jax
import jax.numpy as jnp
from jax import lax
from jax.experimental import pallas as pl
from jax.experimental.pallas import tpu as pltpu
from jax.experimental.pallas import tpu_sc as plsc

N = 10000
NP = 10240
E_REAL = 650000
EPAD = 655360
CHUNK = 128
HEADS = 4
HID = 64
F32 = jnp.float32
I32 = jnp.int32


def _elu(x):
    return jnp.where(x > 0, x, jnp.exp(x) - 1.0)


def _leaky(x):
    return jnp.where(x > 0, x, 0.2 * x)


def _a1_body(x_ref, w_ref, aat_ref, h0_ref, h1_ref, h2_ref, h3_ref, aux_ref):
    h = jnp.dot(x_ref[...], w_ref[...], preferred_element_type=F32)
    h0_ref[...] = h[:, 0:64]
    h1_ref[...] = h[:, 64:128]
    h2_ref[...] = h[:, 128:192]
    h3_ref[...] = h[:, 192:256]
    aux_ref[...] = jnp.dot(h, aat_ref[...], preferred_element_type=F32)


def _tc_layer1(xpad, W1, Aat1):
    bn = 1024
    hspec = pl.BlockSpec((bn, 64), lambda i: (i, 0))
    hshape = jax.ShapeDtypeStruct((NP, 64), F32)
    return pl.pallas_call(
        _a1_body,
        grid=(NP // bn,),
        in_specs=[
            pl.BlockSpec((bn, 16), lambda i: (i, 0)),
            pl.BlockSpec((16, 256), lambda i: (0, 0)),
            pl.BlockSpec((256, 8), lambda i: (0, 0)),
        ],
        out_specs=[hspec, hspec, hspec, hspec,
                   pl.BlockSpec((bn, 8), lambda i: (i, 0))],
        out_shape=[hshape, hshape, hshape, hshape,
                   jax.ShapeDtypeStruct((NP, 8), F32)],
    )(xpad, W1, Aat1)


def _a_next_body(n0_ref, n1_ref, n2_ref, n3_ref, den_ref, b_ref, w_ref,
                 aat_ref, *out_refs, bn, out_ch):
    b = b_ref[...]
    den = den_ref[...]
    nq = (n0_ref, n1_ref, n2_ref, n3_ref)
    xs = []
    for q in range(4):
        dq = jnp.broadcast_to(den[:, q:q + 1] + 1e-16, (bn, 64))
        xs.append(_elu(nq[q][...] / dq + b[:, 64 * q:64 * (q + 1)]))
    x = jnp.concatenate(xs, axis=1)
    h = jnp.dot(x, w_ref[...], preferred_element_type=F32)
    if out_ch == 256:
        h0_ref, h1_ref, h2_ref, h3_ref, aux_ref = out_refs
        h0_ref[...] = h[:, 0:64]
        h1_ref[...] = h[:, 64:128]
        h2_ref[...] = h[:, 128:192]
        h3_ref[...] = h[:, 192:256]
    else:
        h0_ref, aux_ref = out_refs
        h0_ref[...] = h
    aux_ref[...] = jnp.dot(h, aat_ref[...], preferred_element_type=F32)


def _tc_layer_next(n0, n1, n2, n3, den, brow, W, Aat, out_ch):
    bn = 1024
    body = functools.partial(_a_next_body, bn=bn, out_ch=out_ch)
    hspec = pl.BlockSpec((bn, 64), lambda i: (i, 0))
    hshape = jax.ShapeDtypeStruct((NP, 64), F32)
    if out_ch == 256:
        out_specs = [hspec, hspec, hspec, hspec,
                     pl.BlockSpec((bn, 8), lambda i: (i, 0))]
        out_shape = [hshape, hshape, hshape, hshape,
                     jax.ShapeDtypeStruct((NP, 8), F32)]
    else:
        out_specs = [hspec, pl.BlockSpec((bn, 8), lambda i: (i, 0))]
        out_shape = [hshape, jax.ShapeDtypeStruct((NP, 8), F32)]
    return pl.pallas_call(
        body,
        grid=(NP // bn,),
        in_specs=[
            hspec, hspec, hspec, hspec,
            pl.BlockSpec((bn, 64), lambda i: (i, 0)),
            pl.BlockSpec((1, 256), lambda i: (0, 0)),
            pl.BlockSpec((256, W.shape[1]), lambda i: (0, 0)),
            pl.BlockSpec((W.shape[1], 8), lambda i: (0, 0)),
        ],
        out_specs=out_specs,
        out_shape=out_shape,
    )(n0, n1, n2, n3, den, brow, W, Aat)


def _c_body(na_ref, nb_ref, da_ref, db_ref, b_ref, ne_ref, g_ref):
    i = pl.program_id(0)
    bn = 1000
    den = (da_ref[...][:, 0:1] + db_ref[...][:, 0:1]) + 1e-16
    ne = (na_ref[...] + nb_ref[...]) / jnp.broadcast_to(den, (bn, 64)) \
        + b_ref[...]
    ne_ref[...] = ne

    @pl.when(i == 0)
    def _():
        g_ref[...] = jnp.zeros((1, 64), F32)

    g_ref[...] += jnp.sum(ne, axis=0, keepdims=True) * (1.0 / N)


def _tc_final(num3A, num3B, den3A, den3B, b3row):
    bn = 1000
    return pl.pallas_call(
        _c_body,
        grid=(N // bn,),
        in_specs=[
            pl.BlockSpec((bn, 64), lambda i: (i, 0)),
            pl.BlockSpec((bn, 64), lambda i: (i, 0)),
            pl.BlockSpec((bn, 64), lambda i: (i, 0)),
            pl.BlockSpec((bn, 64), lambda i: (i, 0)),
            pl.BlockSpec((1, 64), lambda i: (0, 0)),
        ],
        out_specs=[
            pl.BlockSpec((bn, 64), lambda i: (i, 0)),
            pl.BlockSpec((1, 64), lambda i: (0, 0)),
        ],
        out_shape=[
            jax.ShapeDtypeStruct((N, 64), F32),
            jax.ShapeDtypeStruct((1, 64), F32),
        ],
    )(num3A, num3B, den3A, den3B, b3row)


def _splat(v):
    return lax.broadcast_in_dim(jnp.asarray(v, I32), (16,), ())


def _sc_edge_body(src_ref, dst_ref, auxs_ref, auxd_ref, asmax_ref, h_ref,
                  num_ref, den_ref,
                  idx_sB, idx_dB, auxs, auxdm, rows,
                  exflat, asbuf,
                  exrow, zbuf, zbuf_d, acc_sh, den_sh,
                  gsem0, gsem1, gsem2, gsem3, ssem0, ssem1, ssem2, ssem3,
                  *, qj):
    cid = lax.axis_index("c")
    sid = lax.axis_index("s")
    chan = 64
    nlanes = chan // 16

    if qj is None:
        n_tiles = 32
        tile_id = sid * 2 + cid
        h_off = jnp.asarray(0, I32)
        head_base = jnp.asarray(0, I32)
    else:
        n_tiles = 16
        tile_id = sid
        h_off = (2 * qj + cid) * NP
        head_base = 2 * qj + cid

    epw = EPAD // n_tiles
    nch = epw // CHUNK
    SUP = 32
    nsup = nch // SUP
    tile_row0 = tile_id * nch
    gsems = (gsem0, gsem1, gsem2, gsem3)
    ssems = (ssem0, ssem1, ssem2, ssem3)

    def zrow(r, _):
        for p in range(nlanes):
            zbuf[r, pl.ds(16 * p, 16)] = jnp.zeros((16,), F32)
        zbuf_d[r, pl.ds(0, 16)] = jnp.zeros((16,), F32)
        return 0
    lax.fori_loop(0, 32, zrow, 0)
    rows_per_tile = NP // 16
    r0 = sid * rows_per_tile

    def zcp(j, _):
        pltpu.sync_copy(zbuf, acc_sh.at[pl.ds(r0 + j * 32, 32)])
        pltpu.sync_copy(zbuf_d, den_sh.at[pl.ds(r0 + j * 32, 32)])
        return 0
    lax.fori_loop(0, rows_per_tile // 32, zcp, 0)

    plsc.subcore_barrier()

    pltpu.sync_copy(asmax_ref, asbuf)
    vasmax = asbuf[pl.ds(0, 16)]

    def issue_gathers(j, b):
        pltpu.async_copy(auxs_ref.at[idx_sB.at[j]], auxs.at[b], gsems[b])
        pltpu.async_copy(auxd_ref.at[idx_dB.at[j]], auxdm.at[b], gsems[b])
        pltpu.async_copy(h_ref.at[idx_sB.at[j]], rows.at[b], gsems[b])

    def wait_gathers(j, b):
        pltpu.make_async_copy(auxs_ref.at[idx_sB.at[j]], auxs.at[b],
                              gsems[b]).wait()
        pltpu.make_async_copy(auxd_ref.at[idx_dB.at[j]], auxdm.at[b],
                              gsems[b]).wait()
        pltpu.make_async_copy(h_ref.at[idx_sB.at[j]], rows.at[b],
                              gsems[b]).wait()

    def issue_scatters(j, b):
        pltpu.async_copy(rows.at[b], acc_sh.at[idx_dB.at[j]], ssems[b],
                         add=True)
        pltpu.async_copy(exrow.at[b], den_sh.at[idx_dB.at[j]], ssems[b],
                         add=True)

    def wait_scatters(j, b):
        pltpu.make_async_copy(rows.at[b], acc_sh.at[idx_dB.at[j]],
                              ssems[b]).wait()
        pltpu.make_async_copy(exrow.at[b], den_sh.at[idx_dB.at[j]],
                              ssems[b]).wait()

    def compute(b):
        def mul_body(e, _):
            va = auxs[b, e, pl.ds(0, 16)]
            vd = auxdm[b, e, pl.ds(0, 16)]
            z2 = vasmax + vd
            vm = jnp.maximum(z2, 0.2 * z2)
            t = va + vd
            t = jnp.maximum(t, 0.2 * t)
            ex16 = jnp.exp(t - vm)
            exrow[b, e, pl.ds(0, 16)] = ex16
            exflat[pl.ds(16 * e, 16)] = ex16
            sf = plsc.load_gather(exflat, [_splat(16 * e + head_base)])
            for p in range(4):
                c0 = 16 * p
                rows[b, e, pl.ds(c0, 16)] = rows[b, e, pl.ds(c0, 16)] * sf
            return 0
        lax.fori_loop(0, CHUNK, mul_body, 0)

    def super_body(S, _):
        row0 = tile_row0 + S * SUP
        pltpu.sync_copy(src_ref.at[pl.ds(row0, SUP)], idx_sB)
        pltpu.sync_copy(dst_ref.at[pl.ds(row0, SUP)], idx_dB)

        if qj is not None:
            def hx(r, _):
                for k in range(8):
                    idx_sB[r, pl.ds(16 * k, 16)] = \
                        idx_sB[r, pl.ds(16 * k, 16)] + h_off
                return 0
            lax.fori_loop(0, SUP, hx, 0)

        issue_gathers(0, 0)
        issue_gathers(1, 1)
        for j in range(SUP):
            b = j % 4
            if j + 2 < SUP:
                if j >= 2:
                    wait_scatters(j - 2, (j - 2) % 4)
                issue_gathers(j + 2, (j + 2) % 4)
            wait_gathers(j, b)
            compute(b)
            issue_scatters(j, b)
        for j in range(SUP - 4, SUP):
            wait_scatters(j, j % 4)
        return 0
    lax.fori_loop(0, nsup, super_body, 0)
    plsc.subcore_barrier()

    out_base = cid * NP + r0
    rbuf = rows.at[0]

    def wb(j, _):
        pltpu.sync_copy(acc_sh.at[pl.ds(r0 + j * 64, 64)],
                        rbuf.at[pl.ds(0, 64)])
        pltpu.sync_copy(rbuf.at[pl.ds(0, 64)],
                        num_ref.at[pl.ds(out_base + j * 64, 64)])
        return 0
    lax.fori_loop(0, rows_per_tile // 64, wb, 0)

    dstage = rows.at[0]

    def wbd(j, _):
        pltpu.sync_copy(den_sh.at[pl.ds(r0 + j * 128, 128)],
                        dstage.at[pl.ds(0, 128), pl.ds(0, 16)])
        pltpu.sync_copy(dstage,
                        den_ref.at[pl.ds(out_base + j * 128, 128)])
        return 0
    lax.fori_loop(0, rows_per_tile // 128, wbd, 0)


def _make_sc_edge(qj):
    mesh = plsc.VectorSubcoreMesh(core_axis_name="c", subcore_axis_name="s",
                                  num_cores=2, num_subcores=16)
    body = functools.partial(_sc_edge_body, qj=qj)
    chan = 64
    return pl.kernel(
        body,
        compiler_params=pltpu.CompilerParams(needs_layout_passes=False,
                                             use_tc_tiling_on_sc=False),
        out_type=(
            jax.ShapeDtypeStruct((2 * NP, chan), F32),
            jax.ShapeDtypeStruct((2 * NP, chan), F32),
        ),
        mesh=mesh,
        scratch_types=[
            pltpu.VMEM((32, CHUNK), I32),
            pltpu.VMEM((32, CHUNK), I32),
            pltpu.VMEM((4, CHUNK, 16), F32),
            pltpu.VMEM((4, CHUNK, 16), F32),
            pltpu.VMEM((4, CHUNK, chan), F32),
            pltpu.VMEM((CHUNK * 16,), F32),
            pltpu.VMEM((16,), F32),
            pltpu.VMEM((4, CHUNK, 16), F32),
            pltpu.VMEM((32, chan), F32),
            pltpu.VMEM((32, 16), F32),
            pltpu.VMEM_SHARED((NP, chan), F32),
            pltpu.VMEM_SHARED((NP, 16), F32),
            pltpu.SemaphoreType.DMA,
            pltpu.SemaphoreType.DMA,
            pltpu.SemaphoreType.DMA,
            pltpu.SemaphoreType.DMA,
            pltpu.SemaphoreType.DMA,
            pltpu.SemaphoreType.DMA,
            pltpu.SemaphoreType.DMA,
            pltpu.SemaphoreType.DMA,
        ],
    )


def _head_mats(a_src, a_dst, heads, C):
    k = heads * C
    eye = jnp.eye(heads, dtype=F32)
    Asrc = (eye[:, None, :] * a_src[0][:, :, None]).reshape(k, heads)
    Adst = (eye[:, None, :] * a_dst[0][:, :, None]).reshape(k, heads)
    if heads < 4:
        Asrc = jnp.pad(Asrc, ((0, 0), (0, 4 - heads)))
        Adst = jnp.pad(Adst, ((0, 0), (0, 4 - heads)))
    return jnp.concatenate([Asrc, Adst], axis=1)


def _aux_tables(aux8, rep):
    as_ = aux8[:, :4]
    ad_ = aux8[:, 4:8]
    asmax = jnp.max(as_, axis=0)
    auxS = jnp.tile(as_, (1, 4))
    if rep:
        auxS = jnp.concatenate([auxS] * 4, axis=0)
    auxD = jnp.tile(ad_, (1, 4))
    asmax_row = jnp.tile(asmax, 4)
    return auxS, auxD, asmax_row


def kernel(x, edge_index, W1, a_src1, a_dst1, b1, W2, a_src2, a_dst2, b2,
           W3, a_src3, a_dst3, b3):
    loop = jnp.arange(N, dtype=jnp.int32)
    src = jnp.concatenate([edge_index[0].astype(I32), loop])
    dst = jnp.concatenate([edge_index[1].astype(I32), loop])
    padv = jnp.full((EPAD - E_REAL,), N, I32)
    src = jnp.concatenate([src, padv]).reshape(EPAD // CHUNK, CHUNK)
    dst = jnp.concatenate([dst, padv]).reshape(EPAD // CHUNK, CHUNK)
    xpad = jnp.pad(x, ((0, NP - N), (0, 0)))

    Aat1 = _head_mats(a_src1, a_dst1, HEADS, HID)
    Aat2 = _head_mats(a_src2, a_dst2, HEADS, HID)
    Aat3 = _head_mats(a_src3, a_dst3, 1, HID)
    b1r = b1.reshape(1, 256)
    b2r = b2.reshape(1, 256)
    b3r = b3.reshape(1, 64)

    scq0 = _make_sc_edge(0)
    scq1 = _make_sc_edge(1)
    sc3 = _make_sc_edge(None)

    h0, h1, h2, h3_, aux8 = _tc_layer1(xpad, W1, Aat1)
    auxS, auxD, amr = _aux_tables(aux8, True)
    h_all = jnp.concatenate([h0, h1, h2, h3_], axis=0)
    n01, d01 = scq0(src, dst, auxS, auxD, amr, h_all)
    n23, _ = scq1(src, dst, auxS, auxD, amr, h_all)

    h0, h1, h2, h3_, aux8 = _tc_layer_next(
        n01[:NP], n01[NP:], n23[:NP], n23[NP:], d01[:NP], b1r, W2, Aat2, 256)
    auxS, auxD, amr = _aux_tables(aux8, True)
    h_all = jnp.concatenate([h0, h1, h2, h3_], axis=0)
    n01, d01 = scq0(src, dst, auxS, auxD, amr, h_all)
    n23, _ = scq1(src, dst, auxS, auxD, amr, h_all)

    hL3, aux8 = _tc_layer_next(
        n01[:NP], n01[NP:], n23[:NP], n23[NP:], d01[:NP], b2r, W3, Aat3, 64)
    auxS, auxD, amr = _aux_tables(aux8, False)
    num3, den3 = sc3(src, dst, auxS, auxD, amr, hL3)

    node_embeddings, graph_embedding = _tc_final(
        num3[:NP], num3[NP:], den3[:NP], den3[NP:], b3r)
    return (node_embeddings, graph_embedding)


# speedup vs baseline: 38.0488x; 1.0606x over previous
import functools

import jax
import jax.numpy as jnp
from jax import lax
from jax.experimental import pallas as pl
from jax.experimental.pallas import tpu as pltpu
from jax.experimental.pallas import tpu_sc as plsc

N = 10000
NP = 10240
E_REAL = 650000
EPAD = 655360
CHUNK = 128
HEADS = 4
HID = 64
F32 = jnp.float32
I32 = jnp.int32


def _elu(x):
    return jnp.where(x > 0, x, jnp.exp(x) - 1.0)


def _leaky(x):
    return jnp.where(x > 0, x, 0.2 * x)


def _a1_body(x_ref, w_ref, aat_ref, h0_ref, h1_ref, h2_ref, h3_ref, aux_ref):
    h = jnp.dot(x_ref[...], w_ref[...], preferred_element_type=F32)
    h0_ref[...] = h[:, 0:64]
    h1_ref[...] = h[:, 64:128]
    h2_ref[...] = h[:, 128:192]
    h3_ref[...] = h[:, 192:256]
    aux_ref[...] = jnp.dot(h, aat_ref[...], preferred_element_type=F32)


def _tc_layer1(xpad, W1, Aat1):
    bn = 1024
    hspec = pl.BlockSpec((bn, 64), lambda i: (i, 0))
    hshape = jax.ShapeDtypeStruct((NP, 64), F32)
    return pl.pallas_call(
        _a1_body,
        grid=(NP // bn,),
        in_specs=[
            pl.BlockSpec((bn, 16), lambda i: (i, 0)),
            pl.BlockSpec((16, 256), lambda i: (0, 0)),
            pl.BlockSpec((256, 8), lambda i: (0, 0)),
        ],
        out_specs=[hspec, hspec, hspec, hspec,
                   pl.BlockSpec((bn, 8), lambda i: (i, 0))],
        out_shape=[hshape, hshape, hshape, hshape,
                   jax.ShapeDtypeStruct((NP, 8), F32)],
    )(xpad, W1, Aat1)


def _a_next_body(n0_ref, n1_ref, n2_ref, n3_ref, den_ref, b_ref, w_ref,
                 aat_ref, *out_refs, bn, out_ch):
    b = b_ref[...]
    den = den_ref[...]
    nq = (n0_ref, n1_ref, n2_ref, n3_ref)
    xs = []
    for q in range(4):
        dq = jnp.broadcast_to(den[:, q:q + 1] + 1e-16, (bn, 64))
        xs.append(_elu(nq[q][...] / dq + b[:, 64 * q:64 * (q + 1)]))
    x = jnp.concatenate(xs, axis=1)
    h = jnp.dot(x, w_ref[...], preferred_element_type=F32)
    if out_ch == 256:
        h0_ref, h1_ref, h2_ref, h3_ref, aux_ref = out_refs
        h0_ref[...] = h[:, 0:64]
        h1_ref[...] = h[:, 64:128]
        h2_ref[...] = h[:, 128:192]
        h3_ref[...] = h[:, 192:256]
    else:
        h0_ref, aux_ref = out_refs
        h0_ref[...] = h
    aux_ref[...] = jnp.dot(h, aat_ref[...], preferred_element_type=F32)


def _tc_layer_next(n0, n1, n2, n3, den, brow, W, Aat, out_ch):
    bn = 1024
    body = functools.partial(_a_next_body, bn=bn, out_ch=out_ch)
    hspec = pl.BlockSpec((bn, 64), lambda i: (i, 0))
    hshape = jax.ShapeDtypeStruct((NP, 64), F32)
    if out_ch == 256:
        out_specs = [hspec, hspec, hspec, hspec,
                     pl.BlockSpec((bn, 8), lambda i: (i, 0))]
        out_shape = [hshape, hshape, hshape, hshape,
                     jax.ShapeDtypeStruct((NP, 8), F32)]
    else:
        out_specs = [hspec, pl.BlockSpec((bn, 8), lambda i: (i, 0))]
        out_shape = [hshape, jax.ShapeDtypeStruct((NP, 8), F32)]
    return pl.pallas_call(
        body,
        grid=(NP // bn,),
        in_specs=[
            hspec, hspec, hspec, hspec,
            pl.BlockSpec((bn, 64), lambda i: (i, 0)),
            pl.BlockSpec((1, 256), lambda i: (0, 0)),
            pl.BlockSpec((256, W.shape[1]), lambda i: (0, 0)),
            pl.BlockSpec((W.shape[1], 8), lambda i: (0, 0)),
        ],
        out_specs=out_specs,
        out_shape=out_shape,
    )(n0, n1, n2, n3, den, brow, W, Aat)


def _c_body(na_ref, nb_ref, da_ref, db_ref, b_ref, ne_ref, g_ref):
    i = pl.program_id(0)
    bn = 1000
    den = (da_ref[...][:, 0:1] + db_ref[...][:, 0:1]) + 1e-16
    ne = (na_ref[...] + nb_ref[...]) / jnp.broadcast_to(den, (bn, 64)) \
        + b_ref[...]
    ne_ref[...] = ne

    @pl.when(i == 0)
    def _():
        g_ref[...] = jnp.zeros((1, 64), F32)

    g_ref[...] += jnp.sum(ne, axis=0, keepdims=True) * (1.0 / N)


def _tc_final(num3A, num3B, den3A, den3B, b3row):
    bn = 1000
    return pl.pallas_call(
        _c_body,
        grid=(N // bn,),
        in_specs=[
            pl.BlockSpec((bn, 64), lambda i: (i, 0)),
            pl.BlockSpec((bn, 64), lambda i: (i, 0)),
            pl.BlockSpec((bn, 64), lambda i: (i, 0)),
            pl.BlockSpec((bn, 64), lambda i: (i, 0)),
            pl.BlockSpec((1, 64), lambda i: (0, 0)),
        ],
        out_specs=[
            pl.BlockSpec((bn, 64), lambda i: (i, 0)),
            pl.BlockSpec((1, 64), lambda i: (0, 0)),
        ],
        out_shape=[
            jax.ShapeDtypeStruct((N, 64), F32),
            jax.ShapeDtypeStruct((1, 64), F32),
        ],
    )(num3A, num3B, den3A, den3B, b3row)


def _splat(v):
    return lax.broadcast_in_dim(jnp.asarray(v, I32), (16,), ())


def _sc_edge_body(src_ref, dst_ref, auxs_ref, auxd_ref, asmax_ref, h_ref,
                  num_ref, den_ref,
                  idx_sB, idx_dB, auxs, auxdm, rows,
                  asbuf,
                  exrow, zbuf, zbuf_d, acc_sh, den_sh,
                  gsem0, gsem1, gsem2, gsem3, ssem0, ssem1, ssem2, ssem3,
                  *, qj):
    cid = lax.axis_index("c")
    sid = lax.axis_index("s")
    chan = 64
    nlanes = chan // 16

    if qj is None:
        n_tiles = 32
        tile_id = sid * 2 + cid
        h_off = jnp.asarray(0, I32)
        head_base = jnp.asarray(0, I32)
    else:
        n_tiles = 16
        tile_id = sid
        h_off = (2 * qj + cid) * NP
        head_base = 2 * qj + cid

    epw = EPAD // n_tiles
    nch = epw // CHUNK
    SUP = 32
    nsup = nch // SUP
    tile_row0 = tile_id * nch
    gsems = (gsem0, gsem1, gsem2, gsem3)
    ssems = (ssem0, ssem1, ssem2, ssem3)

    def zrow(r, _):
        for p in range(nlanes):
            zbuf[r, pl.ds(16 * p, 16)] = jnp.zeros((16,), F32)
        zbuf_d[r, pl.ds(0, 16)] = jnp.zeros((16,), F32)
        return 0
    lax.fori_loop(0, 32, zrow, 0)
    rows_per_tile = NP // 16
    r0 = sid * rows_per_tile

    def zcp(j, _):
        pltpu.sync_copy(zbuf, acc_sh.at[pl.ds(r0 + j * 32, 32)])
        pltpu.sync_copy(zbuf_d, den_sh.at[pl.ds(r0 + j * 32, 32)])
        return 0
    lax.fori_loop(0, rows_per_tile // 32, zcp, 0)

    plsc.subcore_barrier()

    pltpu.sync_copy(asmax_ref, asbuf)
    vasmax = asbuf[pl.ds(0, 16)]

    def issue_gathers(j, b):
        pltpu.async_copy(auxs_ref.at[idx_sB.at[j]], auxs.at[b], gsems[b])
        pltpu.async_copy(auxd_ref.at[idx_dB.at[j]], auxdm.at[b], gsems[b])
        pltpu.async_copy(h_ref.at[idx_sB.at[j]], rows.at[b], gsems[b])

    def wait_gathers(j, b):
        pltpu.make_async_copy(auxs_ref.at[idx_sB.at[j]], auxs.at[b],
                              gsems[b]).wait()
        pltpu.make_async_copy(auxd_ref.at[idx_dB.at[j]], auxdm.at[b],
                              gsems[b]).wait()
        pltpu.make_async_copy(h_ref.at[idx_sB.at[j]], rows.at[b],
                              gsems[b]).wait()

    def issue_scatters(j, b):
        pltpu.async_copy(rows.at[b], acc_sh.at[idx_dB.at[j]], ssems[b],
                         add=True)
        pltpu.async_copy(exrow.at[b], den_sh.at[idx_dB.at[j]], ssems[b],
                         add=True)

    def wait_scatters(j, b):
        pltpu.make_async_copy(rows.at[b], acc_sh.at[idx_dB.at[j]],
                              ssems[b]).wait()
        pltpu.make_async_copy(exrow.at[b], den_sh.at[idx_dB.at[j]],
                              ssems[b]).wait()

    hb_idx = _splat(head_base)

    def compute(b):
        def one_edge(e):
            va = auxs[b, e, pl.ds(0, 16)]
            vd = auxdm[b, e, pl.ds(0, 16)]
            z2 = vasmax + vd
            vm = jnp.maximum(z2, 0.2 * z2)
            t = va + vd
            t = jnp.maximum(t, 0.2 * t)
            ex16 = jnp.exp(t - vm)
            exrow[b, e, pl.ds(0, 16)] = ex16
            sf = lax.gather(
                ex16, hb_idx[:, None],
                dimension_numbers=lax.GatherDimensionNumbers(
                    offset_dims=(), collapsed_slice_dims=(0,),
                    start_index_map=(0,)),
                slice_sizes=(1,),
                mode=lax.GatherScatterMode.PROMISE_IN_BOUNDS)
            for p in range(4):
                c0 = 16 * p
                rows[b, e, pl.ds(c0, 16)] = rows[b, e, pl.ds(c0, 16)] * sf

        def mul_body(e2, _):
            one_edge(2 * e2)
            one_edge(2 * e2 + 1)
            return 0
        lax.fori_loop(0, CHUNK // 2, mul_body, 0)

    def super_body(S, _):
        row0 = tile_row0 + S * SUP
        pltpu.sync_copy(src_ref.at[pl.ds(row0, SUP)], idx_sB)
        pltpu.sync_copy(dst_ref.at[pl.ds(row0, SUP)], idx_dB)

        if qj is not None:
            def hx(r, _):
                for k in range(8):
                    idx_sB[r, pl.ds(16 * k, 16)] = \
                        idx_sB[r, pl.ds(16 * k, 16)] + h_off
                return 0
            lax.fori_loop(0, SUP, hx, 0)

        issue_gathers(0, 0)
        issue_gathers(1, 1)
        for j in range(SUP):
            b = j % 4
            if j + 2 < SUP:
                if j >= 2:
                    wait_scatters(j - 2, (j - 2) % 4)
                issue_gathers(j + 2, (j + 2) % 4)
            wait_gathers(j, b)
            compute(b)
            issue_scatters(j, b)
        for j in range(SUP - 4, SUP):
            wait_scatters(j, j % 4)
        return 0
    lax.fori_loop(0, nsup, super_body, 0)
    plsc.subcore_barrier()

    out_base = cid * NP + r0
    rbuf = rows.at[0]

    def wb(j, _):
        pltpu.sync_copy(acc_sh.at[pl.ds(r0 + j * 64, 64)],
                        rbuf.at[pl.ds(0, 64)])
        pltpu.sync_copy(rbuf.at[pl.ds(0, 64)],
                        num_ref.at[pl.ds(out_base + j * 64, 64)])
        return 0
    lax.fori_loop(0, rows_per_tile // 64, wb, 0)

    dstage = rows.at[0]

    def wbd(j, _):
        pltpu.sync_copy(den_sh.at[pl.ds(r0 + j * 128, 128)],
                        dstage.at[pl.ds(0, 128), pl.ds(0, 16)])
        pltpu.sync_copy(dstage,
                        den_ref.at[pl.ds(out_base + j * 128, 128)])
        return 0
    lax.fori_loop(0, rows_per_tile // 128, wbd, 0)


def _make_sc_edge(qj):
    mesh = plsc.VectorSubcoreMesh(core_axis_name="c", subcore_axis_name="s",
                                  num_cores=2, num_subcores=16)
    body = functools.partial(_sc_edge_body, qj=qj)
    chan = 64
    return pl.kernel(
        body,
        compiler_params=pltpu.CompilerParams(needs_layout_passes=False,
                                             use_tc_tiling_on_sc=False),
        out_type=(
            jax.ShapeDtypeStruct((2 * NP, chan), F32),
            jax.ShapeDtypeStruct((2 * NP, chan), F32),
        ),
        mesh=mesh,
        scratch_types=[
            pltpu.VMEM((32, CHUNK), I32),
            pltpu.VMEM((32, CHUNK), I32),
            pltpu.VMEM((4, CHUNK, 16), F32),
            pltpu.VMEM((4, CHUNK, 16), F32),
            pltpu.VMEM((4, CHUNK, chan), F32),
            pltpu.VMEM((16,), F32),
            pltpu.VMEM((4, CHUNK, 16), F32),
            pltpu.VMEM((32, chan), F32),
            pltpu.VMEM((32, 16), F32),
            pltpu.VMEM_SHARED((NP, chan), F32),
            pltpu.VMEM_SHARED((NP, 16), F32),
            pltpu.SemaphoreType.DMA,
            pltpu.SemaphoreType.DMA,
            pltpu.SemaphoreType.DMA,
            pltpu.SemaphoreType.DMA,
            pltpu.SemaphoreType.DMA,
            pltpu.SemaphoreType.DMA,
            pltpu.SemaphoreType.DMA,
            pltpu.SemaphoreType.DMA,
        ],
    )


def _head_mats(a_src, a_dst, heads, C):
    k = heads * C
    eye = jnp.eye(heads, dtype=F32)
    Asrc = (eye[:, None, :] * a_src[0][:, :, None]).reshape(k, heads)
    Adst = (eye[:, None, :] * a_dst[0][:, :, None]).reshape(k, heads)
    if heads < 4:
        Asrc = jnp.pad(Asrc, ((0, 0), (0, 4 - heads)))
        Adst = jnp.pad(Adst, ((0, 0), (0, 4 - heads)))
    return jnp.concatenate([Asrc, Adst], axis=1)


def _aux_tables(aux8, rep):
    as_ = aux8[:, :4]
    ad_ = aux8[:, 4:8]
    asmax = jnp.max(as_, axis=0)
    auxS = jnp.tile(as_, (1, 4))
    if rep:
        auxS = jnp.concatenate([auxS] * 4, axis=0)
    auxD = jnp.tile(ad_, (1, 4))
    asmax_row = jnp.tile(asmax, 4)
    return auxS, auxD, asmax_row


def kernel(x, edge_index, W1, a_src1, a_dst1, b1, W2, a_src2, a_dst2, b2,
           W3, a_src3, a_dst3, b3):
    loop = jnp.arange(N, dtype=jnp.int32)
    src = jnp.concatenate([edge_index[0].astype(I32), loop])
    dst = jnp.concatenate([edge_index[1].astype(I32), loop])
    padv = jnp.full((EPAD - E_REAL,), N, I32)
    src = jnp.concatenate([src, padv]).reshape(EPAD // CHUNK, CHUNK)
    dst = jnp.concatenate([dst, padv]).reshape(EPAD // CHUNK, CHUNK)
    xpad = jnp.pad(x, ((0, NP - N), (0, 0)))

    Aat1 = _head_mats(a_src1, a_dst1, HEADS, HID)
    Aat2 = _head_mats(a_src2, a_dst2, HEADS, HID)
    Aat3 = _head_mats(a_src3, a_dst3, 1, HID)
    b1r = b1.reshape(1, 256)
    b2r = b2.reshape(1, 256)
    b3r = b3.reshape(1, 64)

    scq0 = _make_sc_edge(0)
    scq1 = _make_sc_edge(1)
    sc3 = _make_sc_edge(None)

    h0, h1, h2, h3_, aux8 = _tc_layer1(xpad, W1, Aat1)
    auxS, auxD, amr = _aux_tables(aux8, True)
    h_all = jnp.concatenate([h0, h1, h2, h3_], axis=0)
    n01, d01 = scq0(src, dst, auxS, auxD, amr, h_all)
    n23, _ = scq1(src, dst, auxS, auxD, amr, h_all)

    h0, h1, h2, h3_, aux8 = _tc_layer_next(
        n01[:NP], n01[NP:], n23[:NP], n23[NP:], d01[:NP], b1r, W2, Aat2, 256)
    auxS, auxD, amr = _aux_tables(aux8, True)
    h_all = jnp.concatenate([h0, h1, h2, h3_], axis=0)
    n01, d01 = scq0(src, dst, auxS, auxD, amr, h_all)
    n23, _ = scq1(src, dst, auxS, auxD, amr, h_all)

    hL3, aux8 = _tc_layer_next(
        n01[:NP], n01[NP:], n23[:NP], n23[NP:], d01[:NP], b2r, W3, Aat3, 64)
    auxS, auxD, amr = _aux_tables(aux8, False)
    num3, den3 = sc3(src, dst, auxS, auxD, amr, hL3)

    node_embeddings, graph_embedding = _tc_final(
        num3[:NP], num3[NP:], den3[:NP], den3[NP:], b3r)
    return (node_embeddings, graph_embedding)


# speedup vs baseline: 40.5161x; 1.0648x over previous
import functools

import jax
import jax.numpy as jnp
from jax import lax
from jax.experimental import pallas as pl
from jax.experimental.pallas import tpu as pltpu
from jax.experimental.pallas import tpu_sc as plsc

N = 10000
NP = 10240
E_REAL = 650000
EPAD = 655360
CHUNK = 128
HEADS = 4
HID = 64
F32 = jnp.float32
I32 = jnp.int32


def _elu(x):
    return jnp.where(x > 0, x, jnp.exp(x) - 1.0)


def _leaky(x):
    return jnp.where(x > 0, x, 0.2 * x)


def _a1_body(x_ref, w_ref, aat_ref, h0_ref, h1_ref, h2_ref, h3_ref, aux_ref):
    h = jnp.dot(x_ref[...], w_ref[...], preferred_element_type=F32)
    a8 = jnp.dot(h, aat_ref[...], preferred_element_type=F32)
    asr = jnp.tile(a8[:, :4], (1, 4))
    h0_ref[...] = jnp.concatenate([h[:, 0:64], asr], axis=1)
    h1_ref[...] = jnp.concatenate([h[:, 64:128], asr], axis=1)
    h2_ref[...] = jnp.concatenate([h[:, 128:192], asr], axis=1)
    h3_ref[...] = jnp.concatenate([h[:, 192:256], asr], axis=1)
    aux_ref[...] = a8


def _tc_layer1(xpad, W1, Aat1):
    bn = 1024
    hspec = pl.BlockSpec((bn, 80), lambda i: (i, 0))
    hshape = jax.ShapeDtypeStruct((NP, 80), F32)
    return pl.pallas_call(
        _a1_body,
        grid=(NP // bn,),
        in_specs=[
            pl.BlockSpec((bn, 16), lambda i: (i, 0)),
            pl.BlockSpec((16, 256), lambda i: (0, 0)),
            pl.BlockSpec((256, 8), lambda i: (0, 0)),
        ],
        out_specs=[hspec, hspec, hspec, hspec,
                   pl.BlockSpec((bn, 8), lambda i: (i, 0))],
        out_shape=[hshape, hshape, hshape, hshape,
                   jax.ShapeDtypeStruct((NP, 8), F32)],
    )(xpad, W1, Aat1)


def _a_next_body(n0_ref, n1_ref, n2_ref, n3_ref, b_ref, w_ref,
                 aat_ref, *out_refs, bn, out_ch):
    b = b_ref[...]
    nq = (n0_ref, n1_ref, n2_ref, n3_ref)
    xs = []
    for q in range(4):
        blk = nq[q][...]
        dq = jnp.broadcast_to(blk[:, 64 + q:65 + q] + 1e-16, (bn, 64))
        xs.append(_elu(blk[:, :64] / dq + b[:, 64 * q:64 * (q + 1)]))
    x = jnp.concatenate(xs, axis=1)
    h = jnp.dot(x, w_ref[...], preferred_element_type=F32)
    a8 = jnp.dot(h, aat_ref[...], preferred_element_type=F32)
    asr = jnp.tile(a8[:, :4], (1, 4))
    if out_ch == 256:
        h0_ref, h1_ref, h2_ref, h3_ref, aux_ref = out_refs
        h0_ref[...] = jnp.concatenate([h[:, 0:64], asr], axis=1)
        h1_ref[...] = jnp.concatenate([h[:, 64:128], asr], axis=1)
        h2_ref[...] = jnp.concatenate([h[:, 128:192], asr], axis=1)
        h3_ref[...] = jnp.concatenate([h[:, 192:256], asr], axis=1)
    else:
        h0_ref, aux_ref = out_refs
        h0_ref[...] = jnp.concatenate([h, asr], axis=1)
    aux_ref[...] = a8


def _tc_layer_next(n0, n1, n2, n3, brow, W, Aat, out_ch):
    bn = 1024
    body = functools.partial(_a_next_body, bn=bn, out_ch=out_ch)
    hspec = pl.BlockSpec((bn, 80), lambda i: (i, 0))
    hshape = jax.ShapeDtypeStruct((NP, 80), F32)
    if out_ch == 256:
        out_specs = [hspec, hspec, hspec, hspec,
                     pl.BlockSpec((bn, 8), lambda i: (i, 0))]
        out_shape = [hshape, hshape, hshape, hshape,
                     jax.ShapeDtypeStruct((NP, 8), F32)]
    else:
        out_specs = [hspec, pl.BlockSpec((bn, 8), lambda i: (i, 0))]
        out_shape = [hshape, jax.ShapeDtypeStruct((NP, 8), F32)]
    return pl.pallas_call(
        body,
        grid=(NP // bn,),
        in_specs=[
            hspec, hspec, hspec, hspec,
            pl.BlockSpec((1, 256), lambda i: (0, 0)),
            pl.BlockSpec((256, W.shape[1]), lambda i: (0, 0)),
            pl.BlockSpec((W.shape[1], 8), lambda i: (0, 0)),
        ],
        out_specs=out_specs,
        out_shape=out_shape,
    )(n0, n1, n2, n3, brow, W, Aat)


def _c_body(na_ref, nb_ref, b_ref, ne_ref, g_ref):
    i = pl.program_id(0)
    bn = 1000
    na = na_ref[...]
    nb = nb_ref[...]
    den = (na[:, 64:65] + nb[:, 64:65]) + 1e-16
    ne = (na[:, :64] + nb[:, :64]) / jnp.broadcast_to(den, (bn, 64)) \
        + b_ref[...]
    ne_ref[...] = ne

    @pl.when(i == 0)
    def _():
        g_ref[...] = jnp.zeros((1, 64), F32)

    g_ref[...] += jnp.sum(ne, axis=0, keepdims=True) * (1.0 / N)


def _tc_final(num3A, num3B, b3row):
    bn = 1000
    return pl.pallas_call(
        _c_body,
        grid=(N // bn,),
        in_specs=[
            pl.BlockSpec((bn, 80), lambda i: (i, 0)),
            pl.BlockSpec((bn, 80), lambda i: (i, 0)),
            pl.BlockSpec((1, 64), lambda i: (0, 0)),
        ],
        out_specs=[
            pl.BlockSpec((bn, 64), lambda i: (i, 0)),
            pl.BlockSpec((1, 64), lambda i: (0, 0)),
        ],
        out_shape=[
            jax.ShapeDtypeStruct((N, 64), F32),
            jax.ShapeDtypeStruct((1, 64), F32),
        ],
    )(num3A, num3B, b3row)


def _splat(v):
    return lax.broadcast_in_dim(jnp.asarray(v, I32), (16,), ())


def _sc_edge_body(src_ref, dst_ref, auxd_ref, asmax_ref, h_ref,
                  num_ref,
                  idx_sB, idx_dB, auxdm, rows,
                  asbuf,
                  zbuf, acc_sh,
                  gsem0, gsem1, gsem2, gsem3, ssem0, ssem1, ssem2, ssem3,
                  *, qj):
    cid = lax.axis_index("c")
    sid = lax.axis_index("s")
    chan = 80
    nlanes = chan // 16

    if qj is None:
        n_tiles = 32
        tile_id = sid * 2 + cid
        h_off = jnp.asarray(0, I32)
        head_base = jnp.asarray(0, I32)
    else:
        n_tiles = 16
        tile_id = sid
        h_off = (2 * qj + cid) * NP
        head_base = 2 * qj + cid

    epw = EPAD // n_tiles
    nch = epw // CHUNK
    SUP = 32
    nsup = nch // SUP
    tile_row0 = tile_id * nch
    gsems = (gsem0, gsem1, gsem2, gsem3)
    ssems = (ssem0, ssem1, ssem2, ssem3)

    def zrow(r, _):
        for p in range(nlanes):
            zbuf[r, pl.ds(16 * p, 16)] = jnp.zeros((16,), F32)
        return 0
    lax.fori_loop(0, 32, zrow, 0)
    rows_per_tile = NP // 16
    r0 = sid * rows_per_tile

    def zcp(j, _):
        pltpu.sync_copy(zbuf, acc_sh.at[pl.ds(r0 + j * 32, 32)])
        return 0
    lax.fori_loop(0, rows_per_tile // 32, zcp, 0)

    plsc.subcore_barrier()

    pltpu.sync_copy(asmax_ref, asbuf)
    vasmax = asbuf[pl.ds(0, 16)]

    def issue_gathers(j, b):
        pltpu.async_copy(auxd_ref.at[idx_dB.at[j]], auxdm.at[b], gsems[b])
        pltpu.async_copy(h_ref.at[idx_sB.at[j]], rows.at[b], gsems[b])

    def wait_gathers(j, b):
        pltpu.make_async_copy(auxd_ref.at[idx_dB.at[j]], auxdm.at[b],
                              gsems[b]).wait()
        pltpu.make_async_copy(h_ref.at[idx_sB.at[j]], rows.at[b],
                              gsems[b]).wait()

    def issue_scatters(j, b):
        pltpu.async_copy(rows.at[b], acc_sh.at[idx_dB.at[j]], ssems[b],
                         add=True)

    def wait_scatters(j, b):
        pltpu.make_async_copy(rows.at[b], acc_sh.at[idx_dB.at[j]],
                              ssems[b]).wait()

    hb_idx = _splat(head_base)

    def compute(b):
        def one_edge(e):
            va = rows[b, e, pl.ds(64, 16)]
            vd = auxdm[b, e, pl.ds(0, 16)]
            z2 = vasmax + vd
            vm = jnp.maximum(z2, 0.2 * z2)
            t = va + vd
            t = jnp.maximum(t, 0.2 * t)
            ex16 = jnp.exp(t - vm)
            rows[b, e, pl.ds(64, 16)] = ex16
            sf = lax.gather(
                ex16, hb_idx[:, None],
                dimension_numbers=lax.GatherDimensionNumbers(
                    offset_dims=(), collapsed_slice_dims=(0,),
                    start_index_map=(0,)),
                slice_sizes=(1,),
                mode=lax.GatherScatterMode.PROMISE_IN_BOUNDS)
            for p in range(4):
                c0 = 16 * p
                rows[b, e, pl.ds(c0, 16)] = rows[b, e, pl.ds(c0, 16)] * sf

        def mul_body(e2, _):
            one_edge(2 * e2)
            one_edge(2 * e2 + 1)
            return 0
        lax.fori_loop(0, CHUNK // 2, mul_body, 0)

    def super_body(S, _):
        row0 = tile_row0 + S * SUP
        pltpu.sync_copy(src_ref.at[pl.ds(row0, SUP)], idx_sB)
        pltpu.sync_copy(dst_ref.at[pl.ds(row0, SUP)], idx_dB)

        if qj is not None:
            def hx(r, _):
                for k in range(8):
                    idx_sB[r, pl.ds(16 * k, 16)] = \
                        idx_sB[r, pl.ds(16 * k, 16)] + h_off
                return 0
            lax.fori_loop(0, SUP, hx, 0)

        issue_gathers(0, 0)
        issue_gathers(1, 1)
        for j in range(SUP):
            b = j % 4
            if j + 2 < SUP:
                if j >= 2:
                    wait_scatters(j - 2, (j - 2) % 4)
                issue_gathers(j + 2, (j + 2) % 4)
            wait_gathers(j, b)
            compute(b)
            issue_scatters(j, b)
        for j in range(SUP - 4, SUP):
            wait_scatters(j, j % 4)
        return 0
    lax.fori_loop(0, nsup, super_body, 0)
    plsc.subcore_barrier()

    out_base = cid * NP + r0
    rbuf = rows.at[0]

    def wb(j, _):
        pltpu.sync_copy(acc_sh.at[pl.ds(r0 + j * CHUNK, CHUNK)], rbuf)
        pltpu.sync_copy(rbuf,
                        num_ref.at[pl.ds(out_base + j * CHUNK, CHUNK)])
        return 0
    lax.fori_loop(0, rows_per_tile // CHUNK, wb, 0)


def _make_sc_edge(qj):
    mesh = plsc.VectorSubcoreMesh(core_axis_name="c", subcore_axis_name="s",
                                  num_cores=2, num_subcores=16)
    body = functools.partial(_sc_edge_body, qj=qj)
    chan = 80
    return pl.kernel(
        body,
        compiler_params=pltpu.CompilerParams(needs_layout_passes=False,
                                             use_tc_tiling_on_sc=False),
        out_type=jax.ShapeDtypeStruct((2 * NP, chan), F32),
        mesh=mesh,
        scratch_types=[
            pltpu.VMEM((32, CHUNK), I32),
            pltpu.VMEM((32, CHUNK), I32),
            pltpu.VMEM((4, CHUNK, 16), F32),
            pltpu.VMEM((4, CHUNK, chan), F32),
            pltpu.VMEM((16,), F32),
            pltpu.VMEM((32, chan), F32),
            pltpu.VMEM_SHARED((NP, chan), F32),
            pltpu.SemaphoreType.DMA,
            pltpu.SemaphoreType.DMA,
            pltpu.SemaphoreType.DMA,
            pltpu.SemaphoreType.DMA,
            pltpu.SemaphoreType.DMA,
            pltpu.SemaphoreType.DMA,
            pltpu.SemaphoreType.DMA,
            pltpu.SemaphoreType.DMA,
        ],
    )


def _head_mats(a_src, a_dst, heads, C):
    k = heads * C
    eye = jnp.eye(heads, dtype=F32)
    Asrc = (eye[:, None, :] * a_src[0][:, :, None]).reshape(k, heads)
    Adst = (eye[:, None, :] * a_dst[0][:, :, None]).reshape(k, heads)
    if heads < 4:
        Asrc = jnp.pad(Asrc, ((0, 0), (0, 4 - heads)))
        Adst = jnp.pad(Adst, ((0, 0), (0, 4 - heads)))
    return jnp.concatenate([Asrc, Adst], axis=1)


def _aux_tables(aux8):
    as_ = aux8[:, :4]
    ad_ = aux8[:, 4:8]
    asmax = jnp.max(as_, axis=0)
    auxD = jnp.tile(ad_, (1, 4))
    asmax_row = jnp.tile(asmax, 4)
    return auxD, asmax_row


def kernel(x, edge_index, W1, a_src1, a_dst1, b1, W2, a_src2, a_dst2, b2,
           W3, a_src3, a_dst3, b3):
    loop = jnp.arange(N, dtype=jnp.int32)
    src = jnp.concatenate([edge_index[0].astype(I32), loop])
    dst = jnp.concatenate([edge_index[1].astype(I32), loop])
    padv = jnp.full((EPAD - E_REAL,), N, I32)
    src = jnp.concatenate([src, padv]).reshape(EPAD // CHUNK, CHUNK)
    dst = jnp.concatenate([dst, padv]).reshape(EPAD // CHUNK, CHUNK)
    xpad = jnp.pad(x, ((0, NP - N), (0, 0)))

    Aat1 = _head_mats(a_src1, a_dst1, HEADS, HID)
    Aat2 = _head_mats(a_src2, a_dst2, HEADS, HID)
    Aat3 = _head_mats(a_src3, a_dst3, 1, HID)
    b1r = b1.reshape(1, 256)
    b2r = b2.reshape(1, 256)
    b3r = b3.reshape(1, 64)

    scq0 = _make_sc_edge(0)
    scq1 = _make_sc_edge(1)
    sc3 = _make_sc_edge(None)

    h0, h1, h2, h3_, aux8 = _tc_layer1(xpad, W1, Aat1)
    auxD, amr = _aux_tables(aux8)
    h_all = jnp.concatenate([h0, h1, h2, h3_], axis=0)
    n01 = scq0(src, dst, auxD, amr, h_all)
    n23 = scq1(src, dst, auxD, amr, h_all)

    h0, h1, h2, h3_, aux8 = _tc_layer_next(
        n01[:NP], n01[NP:], n23[:NP], n23[NP:], b1r, W2, Aat2, 256)
    auxD, amr = _aux_tables(aux8)
    h_all = jnp.concatenate([h0, h1, h2, h3_], axis=0)
    n01 = scq0(src, dst, auxD, amr, h_all)
    n23 = scq1(src, dst, auxD, amr, h_all)

    hL3, aux8 = _tc_layer_next(
        n01[:NP], n01[NP:], n23[:NP], n23[NP:], b2r, W3, Aat3, 64)
    auxD, amr = _aux_tables(aux8)
    num3 = sc3(src, dst, auxD, amr, hL3)

    node_embeddings, graph_embedding = _tc_final(num3[:NP], num3[NP:], b3r)
    return (node_embeddings, graph_embedding)


# speedup vs baseline: 41.3800x; 1.0213x over previous
import functools

import jax
import jax.numpy as jnp
from jax import lax
from jax.experimental import pallas as pl
from jax.experimental.pallas import tpu as pltpu
from jax.experimental.pallas import tpu_sc as plsc

N = 10000
NP = 10240
E_REAL = 650000
EPAD = 655360
CHUNK = 128
HEADS = 4
HID = 64
F32 = jnp.float32
I32 = jnp.int32


def _elu(x):
    return jnp.where(x > 0, x, jnp.exp(x) - 1.0)


def _leaky(x):
    return jnp.where(x > 0, x, 0.2 * x)


def _write_hall(h, a8, hall_ref):
    asr = jnp.tile(a8[:, :4], (1, 4))
    for q in range(4):
        hall_ref[q] = jnp.concatenate([h[:, 64 * q:64 * (q + 1)], asr],
                                      axis=1)


def _a1_body(x_ref, w_ref, aat_ref, hall_ref, aux_ref):
    h = jnp.dot(x_ref[...], w_ref[...], preferred_element_type=F32)
    a8 = jnp.dot(h, aat_ref[...], preferred_element_type=F32)
    _write_hall(h, a8, hall_ref)
    aux_ref[...] = a8


def _tc_layer1(xpad, W1, Aat1):
    bn = 1024
    hall, aux8 = pl.pallas_call(
        _a1_body,
        grid=(NP // bn,),
        in_specs=[
            pl.BlockSpec((bn, 16), lambda i: (i, 0)),
            pl.BlockSpec((16, 256), lambda i: (0, 0)),
            pl.BlockSpec((256, 8), lambda i: (0, 0)),
        ],
        out_specs=[pl.BlockSpec((4, bn, 80), lambda i: (0, i, 0)),
                   pl.BlockSpec((bn, 8), lambda i: (i, 0))],
        out_shape=[jax.ShapeDtypeStruct((4, NP, 80), F32),
                   jax.ShapeDtypeStruct((NP, 8), F32)],
    )(xpad, W1, Aat1)
    return hall.reshape(4 * NP, 80), aux8


def _a_next_body(n0_ref, n1_ref, n2_ref, n3_ref, b_ref, w_ref,
                 aat_ref, *out_refs, bn, out_ch):
    b = b_ref[...]
    nq = (n0_ref, n1_ref, n2_ref, n3_ref)
    xs = []
    for q in range(4):
        blk = nq[q][...]
        dq = jnp.broadcast_to(blk[:, 64 + q:65 + q] + 1e-16, (bn, 64))
        xs.append(_elu(blk[:, :64] / dq + b[:, 64 * q:64 * (q + 1)]))
    x = jnp.concatenate(xs, axis=1)
    h = jnp.dot(x, w_ref[...], preferred_element_type=F32)
    a8 = jnp.dot(h, aat_ref[...], preferred_element_type=F32)
    if out_ch == 256:
        hall_ref, aux_ref = out_refs
        _write_hall(h, a8, hall_ref)
    else:
        h0_ref, aux_ref = out_refs
        asr = jnp.tile(a8[:, :4], (1, 4))
        h0_ref[...] = jnp.concatenate([h, asr], axis=1)
    aux_ref[...] = a8


def _tc_layer_next(n0, n1, n2, n3, brow, W, Aat, out_ch):
    bn = 1024
    body = functools.partial(_a_next_body, bn=bn, out_ch=out_ch)
    hspec = pl.BlockSpec((bn, 80), lambda i: (i, 0))
    if out_ch == 256:
        out_specs = [pl.BlockSpec((4, bn, 80), lambda i: (0, i, 0)),
                     pl.BlockSpec((bn, 8), lambda i: (i, 0))]
        out_shape = [jax.ShapeDtypeStruct((4, NP, 80), F32),
                     jax.ShapeDtypeStruct((NP, 8), F32)]
    else:
        out_specs = [hspec, pl.BlockSpec((bn, 8), lambda i: (i, 0))]
        out_shape = [jax.ShapeDtypeStruct((NP, 80), F32),
                     jax.ShapeDtypeStruct((NP, 8), F32)]
    return pl.pallas_call(
        body,
        grid=(NP // bn,),
        in_specs=[
            hspec, hspec, hspec, hspec,
            pl.BlockSpec((1, 256), lambda i: (0, 0)),
            pl.BlockSpec((256, W.shape[1]), lambda i: (0, 0)),
            pl.BlockSpec((W.shape[1], 8), lambda i: (0, 0)),
        ],
        out_specs=out_specs,
        out_shape=out_shape,
    )(n0, n1, n2, n3, brow, W, Aat)


def _c_body(na_ref, nb_ref, b_ref, ne_ref, g_ref):
    i = pl.program_id(0)
    bn = 1000
    na = na_ref[...]
    nb = nb_ref[...]
    den = (na[:, 64:65] + nb[:, 64:65]) + 1e-16
    ne = (na[:, :64] + nb[:, :64]) / jnp.broadcast_to(den, (bn, 64)) \
        + b_ref[...]
    ne_ref[...] = ne

    @pl.when(i == 0)
    def _():
        g_ref[...] = jnp.zeros((1, 64), F32)

    g_ref[...] += jnp.sum(ne, axis=0, keepdims=True) * (1.0 / N)


def _tc_final(num3A, num3B, b3row):
    bn = 1000
    return pl.pallas_call(
        _c_body,
        grid=(N // bn,),
        in_specs=[
            pl.BlockSpec((bn, 80), lambda i: (i, 0)),
            pl.BlockSpec((bn, 80), lambda i: (i, 0)),
            pl.BlockSpec((1, 64), lambda i: (0, 0)),
        ],
        out_specs=[
            pl.BlockSpec((bn, 64), lambda i: (i, 0)),
            pl.BlockSpec((1, 64), lambda i: (0, 0)),
        ],
        out_shape=[
            jax.ShapeDtypeStruct((N, 64), F32),
            jax.ShapeDtypeStruct((1, 64), F32),
        ],
    )(num3A, num3B, b3row)


def _splat(v):
    return lax.broadcast_in_dim(jnp.asarray(v, I32), (16,), ())


def _sc_edge_body(src_ref, dst_ref, auxd_ref, asmax_ref, h_ref,
                  num_ref,
                  idx_sB, idx_dB, auxdm, rows,
                  asbuf,
                  zbuf, acc_sh,
                  gsem0, gsem1, gsem2, gsem3, ssem0, ssem1, ssem2, ssem3,
                  *, qj):
    cid = lax.axis_index("c")
    sid = lax.axis_index("s")
    chan = 80
    nlanes = chan // 16

    if qj is None:
        n_tiles = 32
        tile_id = sid * 2 + cid
        h_off = jnp.asarray(0, I32)
        head_base = jnp.asarray(0, I32)
    else:
        n_tiles = 16
        tile_id = sid
        h_off = (2 * qj + cid) * NP
        head_base = 2 * qj + cid

    epw = EPAD // n_tiles
    nch = epw // CHUNK
    SUP = 32
    nsup = nch // SUP
    tile_row0 = tile_id * nch
    gsems = (gsem0, gsem1, gsem2, gsem3)
    ssems = (ssem0, ssem1, ssem2, ssem3)

    def zrow(r, _):
        for p in range(nlanes):
            zbuf[r, pl.ds(16 * p, 16)] = jnp.zeros((16,), F32)
        return 0
    lax.fori_loop(0, 32, zrow, 0)
    rows_per_tile = NP // 16
    r0 = sid * rows_per_tile

    def zcp(j, _):
        pltpu.sync_copy(zbuf, acc_sh.at[pl.ds(r0 + j * 32, 32)])
        return 0
    lax.fori_loop(0, rows_per_tile // 32, zcp, 0)

    plsc.subcore_barrier()

    pltpu.sync_copy(asmax_ref, asbuf)
    vasmax = asbuf[pl.ds(0, 16)]

    def issue_gathers(j, b):
        pltpu.async_copy(auxd_ref.at[idx_dB.at[j]], auxdm.at[b], gsems[b])
        pltpu.async_copy(h_ref.at[idx_sB.at[j]], rows.at[b], gsems[b])

    def wait_gathers(j, b):
        pltpu.make_async_copy(auxd_ref.at[idx_dB.at[j]], auxdm.at[b],
                              gsems[b]).wait()
        pltpu.make_async_copy(h_ref.at[idx_sB.at[j]], rows.at[b],
                              gsems[b]).wait()

    def issue_scatters(j, b):
        pltpu.async_copy(rows.at[b], acc_sh.at[idx_dB.at[j]], ssems[b],
                         add=True)

    def wait_scatters(j, b):
        pltpu.make_async_copy(rows.at[b], acc_sh.at[idx_dB.at[j]],
                              ssems[b]).wait()

    hb_idx = _splat(head_base)

    def compute(b):
        def one_edge(e):
            va = rows[b, e, pl.ds(64, 16)]
            vd = auxdm[b, e, pl.ds(0, 16)]
            z2 = vasmax + vd
            vm = jnp.maximum(z2, 0.2 * z2)
            t = va + vd
            t = jnp.maximum(t, 0.2 * t)
            ex16 = jnp.exp(t - vm)
            rows[b, e, pl.ds(64, 16)] = ex16
            sf = lax.gather(
                ex16, hb_idx[:, None],
                dimension_numbers=lax.GatherDimensionNumbers(
                    offset_dims=(), collapsed_slice_dims=(0,),
                    start_index_map=(0,)),
                slice_sizes=(1,),
                mode=lax.GatherScatterMode.PROMISE_IN_BOUNDS)
            for p in range(4):
                c0 = 16 * p
                rows[b, e, pl.ds(c0, 16)] = rows[b, e, pl.ds(c0, 16)] * sf

        def mul_body(e2, _):
            one_edge(2 * e2)
            one_edge(2 * e2 + 1)
            return 0
        lax.fori_loop(0, CHUNK // 2, mul_body, 0)

    def super_body(S, _):
        row0 = tile_row0 + S * SUP
        pltpu.sync_copy(src_ref.at[pl.ds(row0, SUP)], idx_sB)
        pltpu.sync_copy(dst_ref.at[pl.ds(row0, SUP)], idx_dB)

        if qj is not None:
            def hx(r, _):
                for k in range(8):
                    idx_sB[r, pl.ds(16 * k, 16)] = \
                        idx_sB[r, pl.ds(16 * k, 16)] + h_off
                return 0
            lax.fori_loop(0, SUP, hx, 0)

        issue_gathers(0, 0)
        issue_gathers(1, 1)
        for j in range(SUP):
            b = j % 4
            if j + 2 < SUP:
                if j >= 2:
                    wait_scatters(j - 2, (j - 2) % 4)
                issue_gathers(j + 2, (j + 2) % 4)
            wait_gathers(j, b)
            compute(b)
            issue_scatters(j, b)
        for j in range(SUP - 4, SUP):
            wait_scatters(j, j % 4)
        return 0
    lax.fori_loop(0, nsup, super_body, 0)
    plsc.subcore_barrier()

    out_base = cid * NP + r0
    rbuf = rows.at[0]

    def wb(j, _):
        pltpu.sync_copy(acc_sh.at[pl.ds(r0 + j * CHUNK, CHUNK)], rbuf)
        pltpu.sync_copy(rbuf,
                        num_ref.at[pl.ds(out_base + j * CHUNK, CHUNK)])
        return 0
    lax.fori_loop(0, rows_per_tile // CHUNK, wb, 0)


def _make_sc_edge(qj):
    mesh = plsc.VectorSubcoreMesh(core_axis_name="c", subcore_axis_name="s",
                                  num_cores=2, num_subcores=16)
    body = functools.partial(_sc_edge_body, qj=qj)
    chan = 80
    return pl.kernel(
        body,
        compiler_params=pltpu.CompilerParams(needs_layout_passes=False,
                                             use_tc_tiling_on_sc=False),
        out_type=jax.ShapeDtypeStruct((2 * NP, chan), F32),
        mesh=mesh,
        scratch_types=[
            pltpu.VMEM((32, CHUNK), I32),
            pltpu.VMEM((32, CHUNK), I32),
            pltpu.VMEM((4, CHUNK, 16), F32),
            pltpu.VMEM((4, CHUNK, chan), F32),
            pltpu.VMEM((16,), F32),
            pltpu.VMEM((32, chan), F32),
            pltpu.VMEM_SHARED((NP, chan), F32),
            pltpu.SemaphoreType.DMA,
            pltpu.SemaphoreType.DMA,
            pltpu.SemaphoreType.DMA,
            pltpu.SemaphoreType.DMA,
            pltpu.SemaphoreType.DMA,
            pltpu.SemaphoreType.DMA,
            pltpu.SemaphoreType.DMA,
            pltpu.SemaphoreType.DMA,
        ],
    )


def _head_mats(a_src, a_dst, heads, C):
    k = heads * C
    eye = jnp.eye(heads, dtype=F32)
    Asrc = (eye[:, None, :] * a_src[0][:, :, None]).reshape(k, heads)
    Adst = (eye[:, None, :] * a_dst[0][:, :, None]).reshape(k, heads)
    if heads < 4:
        Asrc = jnp.pad(Asrc, ((0, 0), (0, 4 - heads)))
        Adst = jnp.pad(Adst, ((0, 0), (0, 4 - heads)))
    return jnp.concatenate([Asrc, Adst], axis=1)


def _aux_tables(aux8):
    as_ = aux8[:, :4]
    ad_ = aux8[:, 4:8]
    asmax = jnp.max(as_, axis=0)
    auxD = jnp.tile(ad_, (1, 4))
    asmax_row = jnp.tile(asmax, 4)
    return auxD, asmax_row


def kernel(x, edge_index, W1, a_src1, a_dst1, b1, W2, a_src2, a_dst2, b2,
           W3, a_src3, a_dst3, b3):
    loop = jnp.arange(N, dtype=jnp.int32)
    src = jnp.concatenate([edge_index[0].astype(I32), loop])
    dst = jnp.concatenate([edge_index[1].astype(I32), loop])
    padv = jnp.full((EPAD - E_REAL,), N, I32)
    src = jnp.concatenate([src, padv]).reshape(EPAD // CHUNK, CHUNK)
    dst = jnp.concatenate([dst, padv]).reshape(EPAD // CHUNK, CHUNK)
    xpad = jnp.pad(x, ((0, NP - N), (0, 0)))

    Aat1 = _head_mats(a_src1, a_dst1, HEADS, HID)
    Aat2 = _head_mats(a_src2, a_dst2, HEADS, HID)
    Aat3 = _head_mats(a_src3, a_dst3, 1, HID)
    b1r = b1.reshape(1, 256)
    b2r = b2.reshape(1, 256)
    b3r = b3.reshape(1, 64)

    scq0 = _make_sc_edge(0)
    scq1 = _make_sc_edge(1)
    sc3 = _make_sc_edge(None)

    h_all, aux8 = _tc_layer1(xpad, W1, Aat1)
    auxD, amr = _aux_tables(aux8)
    n01 = scq0(src, dst, auxD, amr, h_all)
    n23 = scq1(src, dst, auxD, amr, h_all)

    h_all, aux8 = _tc_layer_next(
        n01[:NP], n01[NP:], n23[:NP], n23[NP:], b1r, W2, Aat2, 256)
    h_all = h_all.reshape(4 * NP, 80)
    auxD, amr = _aux_tables(aux8)
    n01 = scq0(src, dst, auxD, amr, h_all)
    n23 = scq1(src, dst, auxD, amr, h_all)

    hL3, aux8 = _tc_layer_next(
        n01[:NP], n01[NP:], n23[:NP], n23[NP:], b2r, W3, Aat3, 64)
    auxD, amr = _aux_tables(aux8)
    num3 = sc3(src, dst, auxD, amr, hL3)

    node_embeddings, graph_embedding = _tc_final(num3[:NP], num3[NP:], b3r)
    return (node_embeddings, graph_embedding)


# speedup vs baseline: 41.9970x; 1.0149x over previous
import functools

import jax
import jax.numpy as jnp
from jax import lax
from jax.experimental import pallas as pl
from jax.experimental.pallas import tpu as pltpu
from jax.experimental.pallas import tpu_sc as plsc

N = 10000
NP = 10240
E_REAL = 650000
EPAD = 655360
CHUNK = 128
HEADS = 4
HID = 64
F32 = jnp.float32
I32 = jnp.int32


def _elu(x):
    return jnp.where(x > 0, x, jnp.exp(x) - 1.0)


def _leaky(x):
    return jnp.where(x > 0, x, 0.2 * x)


def _write_hall(h, a8, hall_ref):
    asr = jnp.tile(a8[:, :4], (1, 4))
    for q in range(4):
        hall_ref[q] = jnp.concatenate([h[:, 64 * q:64 * (q + 1)], asr],
                                      axis=1)


def _a1_body(x_ref, w_ref, aat_ref, hall_ref, aux_ref):
    h = jnp.dot(x_ref[...], w_ref[...], preferred_element_type=F32)
    a8 = jnp.dot(h, aat_ref[...], preferred_element_type=F32)
    _write_hall(h, a8, hall_ref)
    aux_ref[...] = a8


def _tc_layer1(xpad, W1, Aat1):
    bn = 1024
    hall, aux8 = pl.pallas_call(
        _a1_body,
        grid=(NP // bn,),
        in_specs=[
            pl.BlockSpec((bn, 16), lambda i: (i, 0)),
            pl.BlockSpec((16, 256), lambda i: (0, 0)),
            pl.BlockSpec((256, 8), lambda i: (0, 0)),
        ],
        out_specs=[pl.BlockSpec((4, bn, 80), lambda i: (0, i, 0)),
                   pl.BlockSpec((bn, 8), lambda i: (i, 0))],
        out_shape=[jax.ShapeDtypeStruct((4, NP, 80), F32),
                   jax.ShapeDtypeStruct((NP, 8), F32)],
    )(xpad, W1, Aat1)
    return hall.reshape(4 * NP, 80), aux8


def _a_next_body(n0_ref, n1_ref, n2_ref, n3_ref, b_ref, w_ref,
                 aat_ref, *out_refs, bn, out_ch):
    b = b_ref[...]
    nq = (n0_ref, n1_ref, n2_ref, n3_ref)
    xs = []
    for q in range(4):
        blk = nq[q][...]
        dq = jnp.broadcast_to(blk[:, 64 + q:65 + q] + 1e-16, (bn, 64))
        xs.append(_elu(blk[:, :64] / dq + b[:, 64 * q:64 * (q + 1)]))
    x = jnp.concatenate(xs, axis=1)
    h = jnp.dot(x, w_ref[...], preferred_element_type=F32)
    a8 = jnp.dot(h, aat_ref[...], preferred_element_type=F32)
    if out_ch == 256:
        hall_ref, aux_ref = out_refs
        _write_hall(h, a8, hall_ref)
    else:
        h0_ref, aux_ref = out_refs
        asr = jnp.tile(a8[:, :4], (1, 4))
        h0_ref[...] = jnp.concatenate([h, asr], axis=1)
    aux_ref[...] = a8


def _tc_layer_next(n0, n1, n2, n3, brow, W, Aat, out_ch):
    bn = 1024
    body = functools.partial(_a_next_body, bn=bn, out_ch=out_ch)
    hspec = pl.BlockSpec((bn, 80), lambda i: (i, 0))
    if out_ch == 256:
        out_specs = [pl.BlockSpec((4, bn, 80), lambda i: (0, i, 0)),
                     pl.BlockSpec((bn, 8), lambda i: (i, 0))]
        out_shape = [jax.ShapeDtypeStruct((4, NP, 80), F32),
                     jax.ShapeDtypeStruct((NP, 8), F32)]
    else:
        out_specs = [hspec, pl.BlockSpec((bn, 8), lambda i: (i, 0))]
        out_shape = [jax.ShapeDtypeStruct((NP, 80), F32),
                     jax.ShapeDtypeStruct((NP, 8), F32)]
    return pl.pallas_call(
        body,
        grid=(NP // bn,),
        in_specs=[
            hspec, hspec, hspec, hspec,
            pl.BlockSpec((1, 256), lambda i: (0, 0)),
            pl.BlockSpec((256, W.shape[1]), lambda i: (0, 0)),
            pl.BlockSpec((W.shape[1], 8), lambda i: (0, 0)),
        ],
        out_specs=out_specs,
        out_shape=out_shape,
    )(n0, n1, n2, n3, brow, W, Aat)


def _c_body(na_ref, nb_ref, b_ref, ne_ref, g_ref):
    i = pl.program_id(0)
    bn = 1000
    na = na_ref[...]
    nb = nb_ref[...]
    den = (na[:, 64:65] + nb[:, 64:65]) + 1e-16
    ne = (na[:, :64] + nb[:, :64]) / jnp.broadcast_to(den, (bn, 64)) \
        + b_ref[...]
    ne_ref[...] = ne

    @pl.when(i == 0)
    def _():
        g_ref[...] = jnp.zeros((1, 64), F32)

    g_ref[...] += jnp.sum(ne, axis=0, keepdims=True) * (1.0 / N)


def _tc_final(num3A, num3B, b3row):
    bn = 1000
    return pl.pallas_call(
        _c_body,
        grid=(N // bn,),
        in_specs=[
            pl.BlockSpec((bn, 80), lambda i: (i, 0)),
            pl.BlockSpec((bn, 80), lambda i: (i, 0)),
            pl.BlockSpec((1, 64), lambda i: (0, 0)),
        ],
        out_specs=[
            pl.BlockSpec((bn, 64), lambda i: (i, 0)),
            pl.BlockSpec((1, 64), lambda i: (0, 0)),
        ],
        out_shape=[
            jax.ShapeDtypeStruct((N, 64), F32),
            jax.ShapeDtypeStruct((1, 64), F32),
        ],
    )(num3A, num3B, b3row)


def _splat(v):
    return lax.broadcast_in_dim(jnp.asarray(v, I32), (16,), ())


def _sc_edge_body(src_ref, dst_ref, auxd_ref, asmax_ref, h_ref,
                  num_ref,
                  idx_sB, idx_dB, auxdm, rows,
                  asbuf,
                  zbuf, acc_sh,
                  gsem0, gsem1, gsem2, gsem3, ssem0, ssem1, ssem2, ssem3,
                  *, qj):
    cid = lax.axis_index("c")
    sid = lax.axis_index("s")
    chan = 80
    nlanes = chan // 16

    if qj is None:
        n_tiles = 32
        tile_id = sid * 2 + cid
        h_off = jnp.asarray(0, I32)
        head_base = jnp.asarray(0, I32)
    else:
        n_tiles = 16
        tile_id = sid
        h_off = (2 * qj + cid) * NP
        head_base = 2 * qj + cid

    epw = EPAD // n_tiles
    nch = epw // CHUNK
    SUP = 32 if qj is None else 64
    nsup = nch // SUP
    tile_row0 = tile_id * nch
    gsems = (gsem0, gsem1, gsem2, gsem3)
    ssems = (ssem0, ssem1, ssem2, ssem3)

    def zrow(r, _):
        for p in range(nlanes):
            zbuf[r, pl.ds(16 * p, 16)] = jnp.zeros((16,), F32)
        return 0
    lax.fori_loop(0, 32, zrow, 0)
    rows_per_tile = NP // 16
    r0 = sid * rows_per_tile

    def zcp(j, _):
        pltpu.sync_copy(zbuf, acc_sh.at[pl.ds(r0 + j * 32, 32)])
        return 0
    lax.fori_loop(0, rows_per_tile // 32, zcp, 0)

    plsc.subcore_barrier()

    pltpu.sync_copy(asmax_ref, asbuf)
    vasmax = asbuf[pl.ds(0, 16)]

    def issue_gathers(j, b):
        pltpu.async_copy(auxd_ref.at[idx_dB.at[j]], auxdm.at[b], gsems[b])
        pltpu.async_copy(h_ref.at[idx_sB.at[j]], rows.at[b], gsems[b])

    def wait_gathers(j, b):
        pltpu.make_async_copy(auxd_ref.at[idx_dB.at[j]], auxdm.at[b],
                              gsems[b]).wait()
        pltpu.make_async_copy(h_ref.at[idx_sB.at[j]], rows.at[b],
                              gsems[b]).wait()

    def issue_scatters(j, b):
        pltpu.async_copy(rows.at[b], acc_sh.at[idx_dB.at[j]], ssems[b],
                         add=True)

    def wait_scatters(j, b):
        pltpu.make_async_copy(rows.at[b], acc_sh.at[idx_dB.at[j]],
                              ssems[b]).wait()

    hb_idx = _splat(head_base)

    def compute(b):
        def one_edge(e):
            va = rows[b, e, pl.ds(64, 16)]
            vd = auxdm[b, e, pl.ds(0, 16)]
            z2 = vasmax + vd
            vm = jnp.maximum(z2, 0.2 * z2)
            t = va + vd
            t = jnp.maximum(t, 0.2 * t)
            ex16 = jnp.exp(t - vm)
            rows[b, e, pl.ds(64, 16)] = ex16
            sf = lax.gather(
                ex16, hb_idx[:, None],
                dimension_numbers=lax.GatherDimensionNumbers(
                    offset_dims=(), collapsed_slice_dims=(0,),
                    start_index_map=(0,)),
                slice_sizes=(1,),
                mode=lax.GatherScatterMode.PROMISE_IN_BOUNDS)
            for p in range(4):
                c0 = 16 * p
                rows[b, e, pl.ds(c0, 16)] = rows[b, e, pl.ds(c0, 16)] * sf

        def mul_body(e2, _):
            one_edge(2 * e2)
            one_edge(2 * e2 + 1)
            return 0
        lax.fori_loop(0, CHUNK // 2, mul_body, 0)

    def super_body(S, _):
        row0 = tile_row0 + S * SUP
        pltpu.sync_copy(src_ref.at[pl.ds(row0, SUP)], idx_sB)
        pltpu.sync_copy(dst_ref.at[pl.ds(row0, SUP)], idx_dB)

        if qj is not None:
            def hx(r, _):
                for k in range(8):
                    idx_sB[r, pl.ds(16 * k, 16)] = \
                        idx_sB[r, pl.ds(16 * k, 16)] + h_off
                return 0
            lax.fori_loop(0, SUP, hx, 0)

        issue_gathers(0, 0)
        issue_gathers(1, 1)
        for j in range(SUP):
            b = j % 4
            if j + 2 < SUP:
                if j >= 2:
                    wait_scatters(j - 2, (j - 2) % 4)
                issue_gathers(j + 2, (j + 2) % 4)
            wait_gathers(j, b)
            compute(b)
            issue_scatters(j, b)
        for j in range(SUP - 4, SUP):
            wait_scatters(j, j % 4)
        return 0
    lax.fori_loop(0, nsup, super_body, 0)
    plsc.subcore_barrier()

    out_base = cid * NP + r0
    rbuf = rows.at[0]

    def wb(j, _):
        pltpu.sync_copy(acc_sh.at[pl.ds(r0 + j * CHUNK, CHUNK)], rbuf)
        pltpu.sync_copy(rbuf,
                        num_ref.at[pl.ds(out_base + j * CHUNK, CHUNK)])
        return 0
    lax.fori_loop(0, rows_per_tile // CHUNK, wb, 0)


def _make_sc_edge(qj):
    mesh = plsc.VectorSubcoreMesh(core_axis_name="c", subcore_axis_name="s",
                                  num_cores=2, num_subcores=16)
    body = functools.partial(_sc_edge_body, qj=qj)
    chan = 80
    return pl.kernel(
        body,
        compiler_params=pltpu.CompilerParams(needs_layout_passes=False,
                                             use_tc_tiling_on_sc=False),
        out_type=jax.ShapeDtypeStruct((2 * NP, chan), F32),
        mesh=mesh,
        scratch_types=[
            pltpu.VMEM((32 if qj is None else 64, CHUNK), I32),
            pltpu.VMEM((32 if qj is None else 64, CHUNK), I32),
            pltpu.VMEM((4, CHUNK, 16), F32),
            pltpu.VMEM((4, CHUNK, chan), F32),
            pltpu.VMEM((16,), F32),
            pltpu.VMEM((32, chan), F32),
            pltpu.VMEM_SHARED((NP, chan), F32),
            pltpu.SemaphoreType.DMA,
            pltpu.SemaphoreType.DMA,
            pltpu.SemaphoreType.DMA,
            pltpu.SemaphoreType.DMA,
            pltpu.SemaphoreType.DMA,
            pltpu.SemaphoreType.DMA,
            pltpu.SemaphoreType.DMA,
            pltpu.SemaphoreType.DMA,
        ],
    )


def _head_mats(a_src, a_dst, heads, C):
    k = heads * C
    eye = jnp.eye(heads, dtype=F32)
    Asrc = (eye[:, None, :] * a_src[0][:, :, None]).reshape(k, heads)
    Adst = (eye[:, None, :] * a_dst[0][:, :, None]).reshape(k, heads)
    if heads < 4:
        Asrc = jnp.pad(Asrc, ((0, 0), (0, 4 - heads)))
        Adst = jnp.pad(Adst, ((0, 0), (0, 4 - heads)))
    return jnp.concatenate([Asrc, Adst], axis=1)


def _aux_tables(aux8):
    as_ = aux8[:, :4]
    ad_ = aux8[:, 4:8]
    asmax = jnp.max(as_, axis=0)
    auxD = jnp.tile(ad_, (1, 4))
    asmax_row = jnp.tile(asmax, 4)
    return auxD, asmax_row


def kernel(x, edge_index, W1, a_src1, a_dst1, b1, W2, a_src2, a_dst2, b2,
           W3, a_src3, a_dst3, b3):
    loop = jnp.arange(N, dtype=jnp.int32)
    src = jnp.concatenate([edge_index[0].astype(I32), loop])
    dst = jnp.concatenate([edge_index[1].astype(I32), loop])
    padv = jnp.full((EPAD - E_REAL,), N, I32)
    src = jnp.concatenate([src, padv]).reshape(EPAD // CHUNK, CHUNK)
    dst = jnp.concatenate([dst, padv]).reshape(EPAD // CHUNK, CHUNK)
    xpad = jnp.pad(x, ((0, NP - N), (0, 0)))

    Aat1 = _head_mats(a_src1, a_dst1, HEADS, HID)
    Aat2 = _head_mats(a_src2, a_dst2, HEADS, HID)
    Aat3 = _head_mats(a_src3, a_dst3, 1, HID)
    b1r = b1.reshape(1, 256)
    b2r = b2.reshape(1, 256)
    b3r = b3.reshape(1, 64)

    scq0 = _make_sc_edge(0)
    scq1 = _make_sc_edge(1)
    sc3 = _make_sc_edge(None)

    h_all, aux8 = _tc_layer1(xpad, W1, Aat1)
    auxD, amr = _aux_tables(aux8)
    n01 = scq0(src, dst, auxD, amr, h_all)
    n23 = scq1(src, dst, auxD, amr, h_all)

    h_all, aux8 = _tc_layer_next(
        n01[:NP], n01[NP:], n23[:NP], n23[NP:], b1r, W2, Aat2, 256)
    h_all = h_all.reshape(4 * NP, 80)
    auxD, amr = _aux_tables(aux8)
    n01 = scq0(src, dst, auxD, amr, h_all)
    n23 = scq1(src, dst, auxD, amr, h_all)

    hL3, aux8 = _tc_layer_next(
        n01[:NP], n01[NP:], n23[:NP], n23[NP:], b2r, W3, Aat3, 64)
    auxD, amr = _aux_tables(aux8)
    num3 = sc3(src, dst, auxD, amr, hL3)

    node_embeddings, graph_embedding = _tc_final(num3[:NP], num3[NP:], b3r)
    return (node_embeddings, graph_embedding)
